# Initial kernel scaffold; baseline (speedup 1.0000x reference)
import jax
import jax.numpy as jnp
from jax.experimental import pallas as pl


def kernel(x_raw, edge_industry, edge_universe, W_enc, b_enc, gamma, beta, W1, att_src1, att_dst1, bias1, W2, att_src2, att_dst2, bias2, Wf1, bf1, Wf2, bf2, Wa, ba):
    raise NotImplementedError("write your pallas kernel here")


# speedup vs baseline: 17.7545x; 17.7545x over previous
import functools

import jax
import jax.numpy as jnp
from jax import lax
from jax.experimental import pallas as pl
from jax.experimental.pallas import tpu as pltpu
from jax.experimental.pallas import tpu_sc as plsc

N = 10000
F_IN = 256
HID = 64
HEADS = 2
D = HEADS * HID

NC = 2
NS = 16
NW = NC * NS
L = 16

NP = 10240
ROWS_PT = NP // NS
CH = 128
SB = 8
BLK = 1000


def _elu(x):
    return jnp.where(x > 0, x, jnp.exp(x) - 1.0)


def _lrelu(x):
    return jnp.maximum(x, 0.2 * x)


def _tc1_body(x_ref, w_ref, b_ref, wa_ref, ba_ref,
              y_ref, stats_ref, aw_ref, fe_ref):
    i = pl.program_id(0)
    x = x_ref[...]
    y = jnp.dot(x, w_ref[...], preferred_element_type=jnp.float32) + b_ref[...]
    y_ref[...] = y
    s = jnp.sum(y, axis=0)
    sq = jnp.sum(y * y, axis=0)
    st = jnp.concatenate([s[None], sq[None], jnp.zeros((6, HID), jnp.float32)], 0)

    @pl.when(i == 0)
    def _():
        stats_ref[...] = st

    @pl.when(i != 0)
    def _():
        stats_ref[...] += st

    logits = jnp.dot(x, wa_ref[...], preferred_element_type=jnp.float32) + ba_ref[...]
    m = jnp.max(logits, axis=1, keepdims=True)
    e = jnp.exp(logits - m)
    w = e / jnp.sum(e, axis=1, keepdims=True)
    aw_ref[...] = w
    fe = jnp.sum(x * w, axis=1)
    fe_ref[...] = jnp.broadcast_to(fe[:, None], (BLK, 8))


def _tc1(x, w_enc, b_enc, wa, ba):
    return pl.pallas_call(
        _tc1_body,
        grid=(N // BLK,),
        in_specs=[
            pl.BlockSpec((BLK, F_IN), lambda i: (i, 0)),
            pl.BlockSpec((F_IN, HID), lambda i: (0, 0)),
            pl.BlockSpec((HID,), lambda i: (0,)),
            pl.BlockSpec((F_IN, F_IN), lambda i: (0, 0)),
            pl.BlockSpec((F_IN,), lambda i: (0,)),
        ],
        out_specs=[
            pl.BlockSpec((BLK, HID), lambda i: (i, 0)),
            pl.BlockSpec((8, HID), lambda i: (0, 0)),
            pl.BlockSpec((BLK, F_IN), lambda i: (i, 0)),
            pl.BlockSpec((BLK, 8), lambda i: (i, 0)),
        ],
        out_shape=[
            jax.ShapeDtypeStruct((N, HID), jnp.float32),
            jax.ShapeDtypeStruct((8, HID), jnp.float32),
            jax.ShapeDtypeStruct((N, F_IN), jnp.float32),
            jax.ShapeDtypeStruct((N, 8), jnp.float32),
        ],
    )(x, w_enc, b_enc, wa, ba)


def _tc2_body(y_ref, stats_ref, g_ref, be_ref, w1_ref, asr_ref, adr_ref,
              c_ref, h_ref, aa_ref, m_ref):
    i = pl.program_id(0)
    st = stats_ref[...]
    mu = st[0] / N
    var = st[1] / N - mu * mu
    inv = lax.rsqrt(var + 1e-5)
    c = _elu((y_ref[...] - mu) * inv * g_ref[...] + be_ref[...])
    c_ref[...] = c
    h = jnp.dot(c, w1_ref[...], preferred_element_type=jnp.float32)
    h_ref[...] = h
    asr = asr_ref[...]
    adr = adr_ref[...]
    as0 = jnp.sum(h[:, :HID] * asr[0], axis=1)
    as1 = jnp.sum(h[:, HID:] * asr[1], axis=1)
    ad0 = jnp.sum(h[:, :HID] * adr[0], axis=1)
    ad1 = jnp.sum(h[:, HID:] * adr[1], axis=1)
    aa_ref[...] = jnp.stack([as0, as1, ad0, ad1], axis=1)
    mblk = jnp.full((8, 128), jnp.maximum(jnp.max(as0), jnp.max(as1)),
                    jnp.float32)

    @pl.when(i == 0)
    def _():
        m_ref[...] = mblk

    @pl.when(i != 0)
    def _():
        m_ref[...] = jnp.maximum(m_ref[...], mblk)


def _tc2(y, stats, gamma, beta, w1, a_src, a_dst):
    return pl.pallas_call(
        _tc2_body,
        grid=(N // BLK,),
        in_specs=[
            pl.BlockSpec((BLK, HID), lambda i: (i, 0)),
            pl.BlockSpec((8, HID), lambda i: (0, 0)),
            pl.BlockSpec((HID,), lambda i: (0,)),
            pl.BlockSpec((HID,), lambda i: (0,)),
            pl.BlockSpec((HID, D), lambda i: (0, 0)),
            pl.BlockSpec((HEADS, HID), lambda i: (0, 0)),
            pl.BlockSpec((HEADS, HID), lambda i: (0, 0)),
        ],
        out_specs=[
            pl.BlockSpec((BLK, HID), lambda i: (i, 0)),
            pl.BlockSpec((BLK, D), lambda i: (i, 0)),
            pl.BlockSpec((BLK, 4), lambda i: (i, 0)),
            pl.BlockSpec((8, 128), lambda i: (0, 0)),
        ],
        out_shape=[
            jax.ShapeDtypeStruct((N, HID), jnp.float32),
            jax.ShapeDtypeStruct((N, D), jnp.float32),
            jax.ShapeDtypeStruct((N, 4), jnp.float32),
            jax.ShapeDtypeStruct((8, 128), jnp.float32),
        ],
    )(y, stats, gamma, beta, w1, a_src, a_dst)


def _gat_post(nd_ref, bias_ref):
    nd = nd_ref[...]
    h0 = nd[0, :, :HID] / nd[0, :, HID:HID + 1]
    h1 = nd[1, :, :HID] / nd[1, :, HID:HID + 1]
    return _elu(jnp.concatenate([h0, h1], axis=1) + bias_ref[...])


def _tc3_body(nd_ref, b1_ref, c_ref, w2_ref, asr_ref, adr_ref,
              cb_ref, h_ref, aa_ref, m_ref):
    i = pl.program_id(0)
    h_i = _gat_post(nd_ref, b1_ref)
    c = c_ref[...]
    cexp = jnp.concatenate([c, c], axis=1)
    cb = cexp - h_i
    cb_ref[...] = cb
    h = jnp.dot(cb, w2_ref[...], preferred_element_type=jnp.float32)
    h_ref[...] = h
    asr = asr_ref[...]
    adr = adr_ref[...]
    as0 = jnp.sum(h[:, :HID] * asr[0], axis=1)
    as1 = jnp.sum(h[:, HID:] * asr[1], axis=1)
    ad0 = jnp.sum(h[:, :HID] * adr[0], axis=1)
    ad1 = jnp.sum(h[:, HID:] * adr[1], axis=1)
    aa_ref[...] = jnp.stack([as0, as1, ad0, ad1], axis=1)
    mblk = jnp.full((8, 128), jnp.maximum(jnp.max(as0), jnp.max(as1)),
                    jnp.float32)

    @pl.when(i == 0)
    def _():
        m_ref[...] = mblk

    @pl.when(i != 0)
    def _():
        m_ref[...] = jnp.maximum(m_ref[...], mblk)


def _tc3(nd, bias1, c, w2, a_src, a_dst):
    return pl.pallas_call(
        _tc3_body,
        grid=(N // BLK,),
        in_specs=[
            pl.BlockSpec((2, BLK, HID + 16), lambda i: (0, i, 0)),
            pl.BlockSpec((D,), lambda i: (0,)),
            pl.BlockSpec((BLK, HID), lambda i: (i, 0)),
            pl.BlockSpec((D, D), lambda i: (0, 0)),
            pl.BlockSpec((HEADS, HID), lambda i: (0, 0)),
            pl.BlockSpec((HEADS, HID), lambda i: (0, 0)),
        ],
        out_specs=[
            pl.BlockSpec((BLK, D), lambda i: (i, 0)),
            pl.BlockSpec((BLK, D), lambda i: (i, 0)),
            pl.BlockSpec((BLK, 4), lambda i: (i, 0)),
            pl.BlockSpec((8, 128), lambda i: (0, 0)),
        ],
        out_shape=[
            jax.ShapeDtypeStruct((N, D), jnp.float32),
            jax.ShapeDtypeStruct((N, D), jnp.float32),
            jax.ShapeDtypeStruct((N, 4), jnp.float32),
            jax.ShapeDtypeStruct((8, 128), jnp.float32),
        ],
    )(nd, bias1, c, w2, a_src, a_dst)


def _tc4_body(nd_ref, b2_ref, c_ref, cb_ref, wf1_ref, bf1_ref,
              wf2_ref, bf2_ref, df_ref):
    h_u = _gat_post(nd_ref, b2_ref)
    cb = cb_ref[...]
    cbu = cb - h_u
    c = c_ref[...]
    cexp = jnp.concatenate([c, c], axis=1)
    wf1 = wf1_ref[...]
    z = (jnp.dot(cexp, wf1[:D], preferred_element_type=jnp.float32)
         + jnp.dot(cb, wf1[D:2 * D], preferred_element_type=jnp.float32)
         + jnp.dot(cbu, wf1[2 * D:], preferred_element_type=jnp.float32)
         + bf1_ref[...])
    z = _lrelu(z)
    df = jnp.sum(z * wf2_ref[...], axis=1) + bf2_ref[0]
    df_ref[...] = jnp.broadcast_to(df[:, None], (BLK, 8))


def _tc4(nd, bias2, c, cb, wf1, bf1, wf2r, bf2):
    return pl.pallas_call(
        _tc4_body,
        grid=(N // BLK,),
        in_specs=[
            pl.BlockSpec((2, BLK, HID + 16), lambda i: (0, i, 0)),
            pl.BlockSpec((D,), lambda i: (0,)),
            pl.BlockSpec((BLK, HID), lambda i: (i, 0)),
            pl.BlockSpec((BLK, D), lambda i: (i, 0)),
            pl.BlockSpec((3 * D, HID), lambda i: (0, 0)),
            pl.BlockSpec((HID,), lambda i: (0,)),
            pl.BlockSpec((1, HID), lambda i: (0, 0)),
            pl.BlockSpec((1,), lambda i: (0,)),
        ],
        out_specs=[pl.BlockSpec((BLK, 8), lambda i: (i, 0))],
        out_shape=[jax.ShapeDtypeStruct((N, 8), jnp.float32)],
    )(nd, bias2, c, cb, wf1, bf1, wf2r, bf2)


def _sc_gat(h_pad, aa2, m2, src3, dst3, nch):
    mesh = plsc.VectorSubcoreMesh(core_axis_name="c", subcore_axis_name="s",
                                  num_cores=NC, num_subcores=NS)

    W = HID + 16

    @functools.partial(
        pl.kernel,
        out_type=jax.ShapeDtypeStruct((NC, NP, W), jnp.float32),
        mesh=mesh,
        compiler_params=pltpu.CompilerParams(needs_layout_passes=False,
                                             use_tc_tiling_on_sc=False),
        scratch_types=[
            pltpu.VMEM((SB, CH), jnp.int32),
            pltpu.VMEM((SB, CH), jnp.int32),
            pltpu.VMEM((NP * 2,), jnp.float32),
            pltpu.VMEM((16,), jnp.float32),
            pltpu.VMEM((CH, D), jnp.float32),
            pltpu.VMEM((CH, W), jnp.float32),
            pltpu.VMEM_SHARED((NP, W), jnp.float32),
        ],
    )
    def k(h_hbm, aa_hbm, m_hbm, src_hbm, dst_hbm, out_hbm,
          srcv, dstv, aav, mv, rowsv, stgv, acc_sh):
        cid = lax.axis_index("c")
        sid = lax.axis_index("s")

        def zero_rows(r, _):
            for cb in range(W // L):
                stgv[r, pl.ds(cb * L, L)] = jnp.zeros((L,), jnp.float32)
            return 0

        lax.fori_loop(0, CH, zero_rows, 0)

        base = sid * ROWS_PT
        for kk in range(ROWS_PT // CH):
            pltpu.sync_copy(stgv, acc_sh.at[pl.ds(base + kk * CH, CH), :])

        pltpu.sync_copy(aa_hbm.at[cid], aav)
        pltpu.sync_copy(m_hbm.at[cid], mv)
        m0 = mv[...][0]
        oh = (lax.iota(jnp.int32, L) == 0).astype(jnp.float32)

        plsc.subcore_barrier()

        def chunk(ci, _):
            pltpu.sync_copy(h_hbm.at[srcv.at[ci]], rowsv)

            def scale_head(hoff):
                def inner():
                    for g in range(CH // L):
                        sg2 = srcv[ci, pl.ds(g * L, L)] * 2
                        dg2 = dstv[ci, pl.ds(g * L, L)] * 2
                        asv = plsc.load_gather(aav, [sg2])
                        adv = plsc.load_gather(aav, [dg2 + 1])
                        w = jnp.exp(_lrelu(asv + adv) - _lrelu(m0 + adv))
                        for j in range(L):
                            e = g * L + j
                            for rb in range(HID // L):
                                stgv[e, pl.ds(rb * L, L)] = (
                                    rowsv[e, pl.ds(hoff + rb * L, L)] * w[j])
                            stgv[e, pl.ds(HID, L)] = oh * w[j]
                return inner

            lax.cond(cid == 0, scale_head(0), scale_head(HID))

            pltpu.sync_copy(stgv, acc_sh.at[dstv.at[ci]], add=True)
            return 0

        def superchunk(sci, _):
            pltpu.sync_copy(src_hbm.at[sid, pl.ds(sci * SB, SB)], srcv)
            pltpu.sync_copy(dst_hbm.at[sid, pl.ds(sci * SB, SB)], dstv)
            lax.fori_loop(0, SB, chunk, 0)
            return 0

        lax.fori_loop(0, nch // SB, superchunk, 0)

        plsc.subcore_barrier()

        for kk in range(ROWS_PT // CH):
            r0 = base + kk * CH
            pltpu.sync_copy(acc_sh.at[pl.ds(r0, CH), :], stgv)
            pltpu.sync_copy(stgv, out_hbm.at[cid, pl.ds(r0, CH), :])

    return k(h_pad, aa2, m2, src3, dst3)


def _prep_edges(ei):
    e = ei.shape[1] + N
    nch = -(-e // (NS * CH))
    nch = -(-nch // SB) * SB
    epad = NS * CH * nch
    loop = jnp.arange(N, dtype=jnp.int32)
    src = jnp.concatenate(
        [ei[0].astype(jnp.int32), loop,
         jnp.full((epad - e,), NP - 1, jnp.int32)])
    dst = jnp.concatenate(
        [ei[1].astype(jnp.int32), loop,
         jnp.full((epad - e,), NP - 1, jnp.int32)])
    return src.reshape(NS, nch, CH), dst.reshape(NS, nch, CH), nch


def kernel(x_raw, edge_industry, edge_universe, W_enc, b_enc, gamma, beta,
           W1, att_src1, att_dst1, bias1, W2, att_src2, att_dst2, bias2,
           Wf1, bf1, Wf2, bf2, Wa, ba):
    y, stats, attn_weights, fe8 = _tc1(x_raw, W_enc, b_enc, Wa, ba)
    c, h1, aa1, m1 = _tc2(y, stats, gamma, beta, W1, att_src1, att_dst1)

    src1, dst1, nch1 = _prep_edges(edge_industry)
    src2, dst2, nch2 = _prep_edges(edge_universe)

    def _sc_inputs(h, aa, m):
        hp = jnp.pad(h, ((0, NP - N), (0, 0)))
        aap = jnp.pad(aa, ((0, NP - N), (0, 0)))
        aa2c = jnp.stack(
            [jnp.stack([aap[:, hd], aap[:, 2 + hd]], axis=-1).reshape(-1)
             for hd in range(HEADS)])
        m2c = jnp.broadcast_to(m[0, :16][None], (2, 16))
        return hp, aa2c, m2c

    h1p, aat1, m1c = _sc_inputs(h1, aa1, m1)
    nd1 = _sc_gat(h1p, aat1, m1c, src1, dst1, nch1)

    cb, h2, aa2, m2 = _tc3(nd1[:, :N], bias1, c, W2, att_src2, att_dst2)

    h2p, aat2, m2c = _sc_inputs(h2, aa2, m2)
    nd2 = _sc_gat(h2p, aat2, m2c, src2, dst2, nch2)

    df8 = _tc4(nd2[:, :N], bias2, c, cb, Wf1, bf1,
               Wf2.reshape(1, HID), bf2)[0]

    deep_factor = df8[:, 0]
    factor_estimate = fe8[:, 0]
    return (deep_factor, factor_estimate, attn_weights)


# speedup vs baseline: 37.0156x; 2.0849x over previous
import functools

import jax
import jax.numpy as jnp
from jax import lax
from jax.experimental import pallas as pl
from jax.experimental.pallas import tpu as pltpu
from jax.experimental.pallas import tpu_sc as plsc

N = 10000
F_IN = 256
HID = 64
HEADS = 2
D = HEADS * HID

NC = 2
NS = 16
NW = NC * NS
L = 16

NP = 10240
ROWS_PT = NP // NS
CH = 128
SB = 8
BLK = 1000


def _elu(x):
    return jnp.where(x > 0, x, jnp.exp(x) - 1.0)


def _lrelu(x):
    return jnp.maximum(x, 0.2 * x)


def _tc1_body(x_ref, w_ref, b_ref, wa_ref, ba_ref,
              y_ref, stats_ref, aw_ref, fe_ref):
    i = pl.program_id(0)
    x = x_ref[...]
    y = jnp.dot(x, w_ref[...], preferred_element_type=jnp.float32) + b_ref[...]
    y_ref[...] = y
    s = jnp.sum(y, axis=0)
    sq = jnp.sum(y * y, axis=0)
    st = jnp.concatenate([s[None], sq[None], jnp.zeros((6, HID), jnp.float32)], 0)

    @pl.when(i == 0)
    def _():
        stats_ref[...] = st

    @pl.when(i != 0)
    def _():
        stats_ref[...] += st

    logits = jnp.dot(x, wa_ref[...], preferred_element_type=jnp.float32) + ba_ref[...]
    m = jnp.max(logits, axis=1, keepdims=True)
    e = jnp.exp(logits - m)
    w = e / jnp.sum(e, axis=1, keepdims=True)
    aw_ref[...] = w
    fe = jnp.sum(x * w, axis=1)
    fe_ref[...] = jnp.broadcast_to(fe[:, None], (BLK, 8))


def _tc1(x, w_enc, b_enc, wa, ba):
    return pl.pallas_call(
        _tc1_body,
        grid=(N // BLK,),
        in_specs=[
            pl.BlockSpec((BLK, F_IN), lambda i: (i, 0)),
            pl.BlockSpec((F_IN, HID), lambda i: (0, 0)),
            pl.BlockSpec((HID,), lambda i: (0,)),
            pl.BlockSpec((F_IN, F_IN), lambda i: (0, 0)),
            pl.BlockSpec((F_IN,), lambda i: (0,)),
        ],
        out_specs=[
            pl.BlockSpec((BLK, HID), lambda i: (i, 0)),
            pl.BlockSpec((8, HID), lambda i: (0, 0)),
            pl.BlockSpec((BLK, F_IN), lambda i: (i, 0)),
            pl.BlockSpec((BLK, 8), lambda i: (i, 0)),
        ],
        out_shape=[
            jax.ShapeDtypeStruct((N, HID), jnp.float32),
            jax.ShapeDtypeStruct((8, HID), jnp.float32),
            jax.ShapeDtypeStruct((N, F_IN), jnp.float32),
            jax.ShapeDtypeStruct((N, 8), jnp.float32),
        ],
    )(x, w_enc, b_enc, wa, ba)


def _tc2_body(y_ref, stats_ref, g_ref, be_ref, w1_ref, asr_ref, adr_ref,
              c_ref, h_ref, aa_ref, m_ref):
    i = pl.program_id(0)
    st = stats_ref[...]
    mu = st[0] / N
    var = st[1] / N - mu * mu
    inv = lax.rsqrt(var + 1e-5)
    c = _elu((y_ref[...] - mu) * inv * g_ref[...] + be_ref[...])
    c_ref[...] = c
    h = jnp.dot(c, w1_ref[...], preferred_element_type=jnp.float32)
    h_ref[...] = h
    asr = asr_ref[...]
    adr = adr_ref[...]
    as0 = jnp.sum(h[:, :HID] * asr[0], axis=1)
    as1 = jnp.sum(h[:, HID:] * asr[1], axis=1)
    ad0 = jnp.sum(h[:, :HID] * adr[0], axis=1)
    ad1 = jnp.sum(h[:, HID:] * adr[1], axis=1)
    aa_ref[...] = jnp.stack([as0, as1, ad0, ad1], axis=1)
    mblk = jnp.full((8, 128), jnp.maximum(jnp.max(as0), jnp.max(as1)),
                    jnp.float32)

    @pl.when(i == 0)
    def _():
        m_ref[...] = mblk

    @pl.when(i != 0)
    def _():
        m_ref[...] = jnp.maximum(m_ref[...], mblk)


def _tc2(y, stats, gamma, beta, w1, a_src, a_dst):
    return pl.pallas_call(
        _tc2_body,
        grid=(N // BLK,),
        in_specs=[
            pl.BlockSpec((BLK, HID), lambda i: (i, 0)),
            pl.BlockSpec((8, HID), lambda i: (0, 0)),
            pl.BlockSpec((HID,), lambda i: (0,)),
            pl.BlockSpec((HID,), lambda i: (0,)),
            pl.BlockSpec((HID, D), lambda i: (0, 0)),
            pl.BlockSpec((HEADS, HID), lambda i: (0, 0)),
            pl.BlockSpec((HEADS, HID), lambda i: (0, 0)),
        ],
        out_specs=[
            pl.BlockSpec((BLK, HID), lambda i: (i, 0)),
            pl.BlockSpec((BLK, D), lambda i: (i, 0)),
            pl.BlockSpec((BLK, 4), lambda i: (i, 0)),
            pl.BlockSpec((8, 128), lambda i: (0, 0)),
        ],
        out_shape=[
            jax.ShapeDtypeStruct((N, HID), jnp.float32),
            jax.ShapeDtypeStruct((N, D), jnp.float32),
            jax.ShapeDtypeStruct((N, 4), jnp.float32),
            jax.ShapeDtypeStruct((8, 128), jnp.float32),
        ],
    )(y, stats, gamma, beta, w1, a_src, a_dst)


def _gat_post(nd_ref, bias_ref):
    nd = nd_ref[...]
    h0 = nd[0, :, :HID] / nd[0, :, HID:HID + 1]
    h1 = nd[1, :, :HID] / nd[1, :, HID:HID + 1]
    return _elu(jnp.concatenate([h0, h1], axis=1) + bias_ref[...])


def _tc3_body(nd_ref, b1_ref, c_ref, w2_ref, asr_ref, adr_ref,
              cb_ref, h_ref, aa_ref, m_ref):
    i = pl.program_id(0)
    h_i = _gat_post(nd_ref, b1_ref)
    c = c_ref[...]
    cexp = jnp.concatenate([c, c], axis=1)
    cb = cexp - h_i
    cb_ref[...] = cb
    h = jnp.dot(cb, w2_ref[...], preferred_element_type=jnp.float32)
    h_ref[...] = h
    asr = asr_ref[...]
    adr = adr_ref[...]
    as0 = jnp.sum(h[:, :HID] * asr[0], axis=1)
    as1 = jnp.sum(h[:, HID:] * asr[1], axis=1)
    ad0 = jnp.sum(h[:, :HID] * adr[0], axis=1)
    ad1 = jnp.sum(h[:, HID:] * adr[1], axis=1)
    aa_ref[...] = jnp.stack([as0, as1, ad0, ad1], axis=1)
    mblk = jnp.full((8, 128), jnp.maximum(jnp.max(as0), jnp.max(as1)),
                    jnp.float32)

    @pl.when(i == 0)
    def _():
        m_ref[...] = mblk

    @pl.when(i != 0)
    def _():
        m_ref[...] = jnp.maximum(m_ref[...], mblk)


def _tc3(nd, bias1, c, w2, a_src, a_dst):
    return pl.pallas_call(
        _tc3_body,
        grid=(N // BLK,),
        in_specs=[
            pl.BlockSpec((2, BLK, HID + 16), lambda i: (0, i, 0)),
            pl.BlockSpec((D,), lambda i: (0,)),
            pl.BlockSpec((BLK, HID), lambda i: (i, 0)),
            pl.BlockSpec((D, D), lambda i: (0, 0)),
            pl.BlockSpec((HEADS, HID), lambda i: (0, 0)),
            pl.BlockSpec((HEADS, HID), lambda i: (0, 0)),
        ],
        out_specs=[
            pl.BlockSpec((BLK, D), lambda i: (i, 0)),
            pl.BlockSpec((BLK, D), lambda i: (i, 0)),
            pl.BlockSpec((BLK, 4), lambda i: (i, 0)),
            pl.BlockSpec((8, 128), lambda i: (0, 0)),
        ],
        out_shape=[
            jax.ShapeDtypeStruct((N, D), jnp.float32),
            jax.ShapeDtypeStruct((N, D), jnp.float32),
            jax.ShapeDtypeStruct((N, 4), jnp.float32),
            jax.ShapeDtypeStruct((8, 128), jnp.float32),
        ],
    )(nd, bias1, c, w2, a_src, a_dst)


def _tc4_body(nd_ref, b2_ref, c_ref, cb_ref, wf1_ref, bf1_ref,
              wf2_ref, bf2_ref, df_ref):
    h_u = _gat_post(nd_ref, b2_ref)
    cb = cb_ref[...]
    cbu = cb - h_u
    c = c_ref[...]
    cexp = jnp.concatenate([c, c], axis=1)
    wf1 = wf1_ref[...]
    z = (jnp.dot(cexp, wf1[:D], preferred_element_type=jnp.float32)
         + jnp.dot(cb, wf1[D:2 * D], preferred_element_type=jnp.float32)
         + jnp.dot(cbu, wf1[2 * D:], preferred_element_type=jnp.float32)
         + bf1_ref[...])
    z = _lrelu(z)
    df = jnp.sum(z * wf2_ref[...], axis=1) + bf2_ref[0]
    df_ref[...] = jnp.broadcast_to(df[:, None], (BLK, 8))


def _tc4(nd, bias2, c, cb, wf1, bf1, wf2r, bf2):
    return pl.pallas_call(
        _tc4_body,
        grid=(N // BLK,),
        in_specs=[
            pl.BlockSpec((2, BLK, HID + 16), lambda i: (0, i, 0)),
            pl.BlockSpec((D,), lambda i: (0,)),
            pl.BlockSpec((BLK, HID), lambda i: (i, 0)),
            pl.BlockSpec((BLK, D), lambda i: (i, 0)),
            pl.BlockSpec((3 * D, HID), lambda i: (0, 0)),
            pl.BlockSpec((HID,), lambda i: (0,)),
            pl.BlockSpec((1, HID), lambda i: (0, 0)),
            pl.BlockSpec((1,), lambda i: (0,)),
        ],
        out_specs=[pl.BlockSpec((BLK, 8), lambda i: (i, 0))],
        out_shape=[jax.ShapeDtypeStruct((N, 8), jnp.float32)],
    )(nd, bias2, c, cb, wf1, bf1, wf2r, bf2)


def _sc_gat(h_pad, aa2, m2, src3, dst3, nch):
    mesh = plsc.VectorSubcoreMesh(core_axis_name="c", subcore_axis_name="s",
                                  num_cores=NC, num_subcores=NS)

    W = HID + 16

    @functools.partial(
        pl.kernel,
        out_type=jax.ShapeDtypeStruct((NC, NP, W), jnp.float32),
        mesh=mesh,
        compiler_params=pltpu.CompilerParams(needs_layout_passes=False,
                                             use_tc_tiling_on_sc=False),
        scratch_types=[
            pltpu.VMEM((SB, CH), jnp.int32),
            pltpu.VMEM((SB, CH), jnp.int32),
            pltpu.VMEM((NP * 2,), jnp.float32),
            pltpu.VMEM((16,), jnp.float32),
            pltpu.VMEM((2, CH, HID), jnp.float32),
            pltpu.VMEM((2, CH, W), jnp.float32),
            pltpu.VMEM_SHARED((NP, W), jnp.float32),
            pltpu.SemaphoreType.DMA,
            pltpu.SemaphoreType.DMA,
            pltpu.SemaphoreType.DMA,
        ],
    )
    def k(h_hbm, aa_hbm, m_hbm, src_hbm, dst_hbm, out_hbm,
          srcv, dstv, aav, mv, rowsv, stgv, acc_sh, gsem, ssem0, ssem1):
        cid = lax.axis_index("c")
        sid = lax.axis_index("s")

        def zero_rows(r, _):
            for cb in range(W // L):
                stgv[0, r, pl.ds(cb * L, L)] = jnp.zeros((L,), jnp.float32)
                stgv[1, r, pl.ds(cb * L, L)] = jnp.zeros((L,), jnp.float32)
            return 0

        lax.fori_loop(0, CH, zero_rows, 0)

        base = sid * ROWS_PT
        for kk in range(ROWS_PT // CH):
            pltpu.sync_copy(stgv.at[0], acc_sh.at[pl.ds(base + kk * CH, CH), :])

        pltpu.sync_copy(aa_hbm.at[cid], aav)
        pltpu.sync_copy(m_hbm.at[cid], mv)
        m0 = mv[...][0]
        oh = (lax.iota(jnp.int32, L) == 0).astype(jnp.float32)

        plsc.subcore_barrier()

        def start_gather(ci, b):
            pltpu.async_copy(h_hbm.at[cid].at[srcv.at[ci]], rowsv.at[b],
                             gsem)

        def wait_gather(b):
            pltpu.make_async_copy(
                h_hbm.at[cid].at[srcv.at[0]], rowsv.at[b], gsem).wait()

        def wait_scatter(b):
            pltpu.make_async_copy(
                stgv.at[b], acc_sh.at[dstv.at[0]],
                ssem0 if b == 0 else ssem1).wait()

        def process(ci, b):
            for g in range(CH // L):
                sg2 = srcv[ci, pl.ds(g * L, L)] * 2
                dg2 = dstv[ci, pl.ds(g * L, L)] * 2
                asv = plsc.load_gather(aav, [sg2])
                adv = plsc.load_gather(aav, [dg2 + 1])
                w = jnp.exp(_lrelu(asv + adv) - _lrelu(m0 + adv))
                for j in range(L):
                    e = g * L + j
                    for rb in range(HID // L):
                        stgv[b, e, pl.ds(rb * L, L)] = (
                            rowsv[b, e, pl.ds(rb * L, L)] * w[j])
                    stgv[b, e, pl.ds(HID, L)] = oh * w[j]

            pltpu.async_copy(stgv.at[b], acc_sh.at[dstv.at[ci]],
                             ssem0 if b == 0 else ssem1, add=True)

        def superchunk(sci, _):
            pltpu.sync_copy(src_hbm.at[sid, pl.ds(sci * SB, SB)], srcv)
            pltpu.sync_copy(dst_hbm.at[sid, pl.ds(sci * SB, SB)], dstv)
            start_gather(0, 0)

            def pair(cj, _):
                ci0 = 2 * cj
                wait_gather(0)
                start_gather(ci0 + 1, 1)

                @pl.when((sci > 0) | (cj > 0))
                def _():
                    wait_scatter(0)

                process(ci0, 0)
                wait_gather(1)

                @pl.when(ci0 + 2 < SB)
                def _():
                    start_gather(ci0 + 2, 0)

                @pl.when((sci > 0) | (cj > 0))
                def _():
                    wait_scatter(1)

                process(ci0 + 1, 1)
                return 0

            lax.fori_loop(0, SB // 2, pair, 0)
            return 0

        lax.fori_loop(0, nch // SB, superchunk, 0)
        wait_scatter(0)
        wait_scatter(1)

        plsc.subcore_barrier()

        for kk in range(ROWS_PT // CH):
            r0 = base + kk * CH
            pltpu.sync_copy(acc_sh.at[pl.ds(r0, CH), :], stgv.at[0])
            pltpu.sync_copy(stgv.at[0], out_hbm.at[cid, pl.ds(r0, CH), :])

    return k(h_pad, aa2, m2, src3, dst3)


def _prep_edges(ei):
    e = ei.shape[1] + N
    nch = -(-e // (NS * CH))
    nch = -(-nch // SB) * SB
    epad = NS * CH * nch
    loop = jnp.arange(N, dtype=jnp.int32)
    src = jnp.concatenate(
        [ei[0].astype(jnp.int32), loop,
         jnp.full((epad - e,), NP - 1, jnp.int32)])
    dst = jnp.concatenate(
        [ei[1].astype(jnp.int32), loop,
         jnp.full((epad - e,), NP - 1, jnp.int32)])
    return src.reshape(NS, nch, CH), dst.reshape(NS, nch, CH), nch


def kernel(x_raw, edge_industry, edge_universe, W_enc, b_enc, gamma, beta,
           W1, att_src1, att_dst1, bias1, W2, att_src2, att_dst2, bias2,
           Wf1, bf1, Wf2, bf2, Wa, ba):
    y, stats, attn_weights, fe8 = _tc1(x_raw, W_enc, b_enc, Wa, ba)
    c, h1, aa1, m1 = _tc2(y, stats, gamma, beta, W1, att_src1, att_dst1)

    src1, dst1, nch1 = _prep_edges(edge_industry)
    src2, dst2, nch2 = _prep_edges(edge_universe)

    def _sc_inputs(h, aa, m):
        hp = jnp.pad(h, ((0, NP - N), (0, 0)))
        hs = jnp.stack([hp[:, :HID], hp[:, HID:]])
        aap = jnp.pad(aa, ((0, NP - N), (0, 0)))
        aa2c = jnp.stack(
            [jnp.stack([aap[:, hd], aap[:, 2 + hd]], axis=-1).reshape(-1)
             for hd in range(HEADS)])
        m2c = jnp.broadcast_to(m[0, :16][None], (2, 16))
        return hs, aa2c, m2c

    h1p, aat1, m1c = _sc_inputs(h1, aa1, m1)
    nd1 = _sc_gat(h1p, aat1, m1c, src1, dst1, nch1)

    cb, h2, aa2, m2 = _tc3(nd1[:, :N], bias1, c, W2, att_src2, att_dst2)

    h2p, aat2, m2c = _sc_inputs(h2, aa2, m2)
    nd2 = _sc_gat(h2p, aat2, m2c, src2, dst2, nch2)

    df8 = _tc4(nd2[:, :N], bias2, c, cb, Wf1, bf1,
               Wf2.reshape(1, HID), bf2)[0]

    deep_factor = df8[:, 0]
    factor_estimate = fe8[:, 0]
    return (deep_factor, factor_estimate, attn_weights)


# speedup vs baseline: 38.3415x; 1.0358x over previous
import functools

import jax
import jax.numpy as jnp
from jax import lax
from jax.experimental import pallas as pl
from jax.experimental.pallas import tpu as pltpu
from jax.experimental.pallas import tpu_sc as plsc

N = 10000
F_IN = 256
HID = 64
HEADS = 2
D = HEADS * HID

NC = 2
NS = 16
NW = NC * NS
L = 16

NP = 10240
ROWS_PT = NP // NS
CH = 128
SB = 8
BLK = 1000


def _elu(x):
    return jnp.where(x > 0, x, jnp.exp(x) - 1.0)


def _lrelu(x):
    return jnp.maximum(x, 0.2 * x)


def _tc1_body(x_ref, w_ref, b_ref, wa_ref, ba_ref,
              y_ref, stats_ref, aw_ref, fe_ref):
    i = pl.program_id(0)
    x = x_ref[...]
    y = jnp.dot(x, w_ref[...], preferred_element_type=jnp.float32) + b_ref[...]
    y_ref[...] = y
    s = jnp.sum(y, axis=0)
    sq = jnp.sum(y * y, axis=0)
    st = jnp.concatenate([s[None], sq[None], jnp.zeros((6, HID), jnp.float32)], 0)

    @pl.when(i == 0)
    def _():
        stats_ref[...] = st

    @pl.when(i != 0)
    def _():
        stats_ref[...] += st

    logits = jnp.dot(x, wa_ref[...], preferred_element_type=jnp.float32) + ba_ref[...]
    m = jnp.max(logits, axis=1, keepdims=True)
    e = jnp.exp(logits - m)
    w = e / jnp.sum(e, axis=1, keepdims=True)
    aw_ref[...] = w
    fe = jnp.sum(x * w, axis=1)
    fe_ref[...] = jnp.broadcast_to(fe[:, None], (BLK, 8))


def _tc1(x, w_enc, b_enc, wa, ba):
    return pl.pallas_call(
        _tc1_body,
        grid=(N // BLK,),
        in_specs=[
            pl.BlockSpec((BLK, F_IN), lambda i: (i, 0)),
            pl.BlockSpec((F_IN, HID), lambda i: (0, 0)),
            pl.BlockSpec((HID,), lambda i: (0,)),
            pl.BlockSpec((F_IN, F_IN), lambda i: (0, 0)),
            pl.BlockSpec((F_IN,), lambda i: (0,)),
        ],
        out_specs=[
            pl.BlockSpec((BLK, HID), lambda i: (i, 0)),
            pl.BlockSpec((8, HID), lambda i: (0, 0)),
            pl.BlockSpec((BLK, F_IN), lambda i: (i, 0)),
            pl.BlockSpec((BLK, 8), lambda i: (i, 0)),
        ],
        out_shape=[
            jax.ShapeDtypeStruct((N, HID), jnp.float32),
            jax.ShapeDtypeStruct((8, HID), jnp.float32),
            jax.ShapeDtypeStruct((N, F_IN), jnp.float32),
            jax.ShapeDtypeStruct((N, 8), jnp.float32),
        ],
    )(x, w_enc, b_enc, wa, ba)


def _tc2_body(y_ref, stats_ref, g_ref, be_ref, w1_ref, asr_ref, adr_ref,
              c_ref, h_ref, aa_ref, m_ref):
    i = pl.program_id(0)
    st = stats_ref[...]
    mu = st[0] / N
    var = st[1] / N - mu * mu
    inv = lax.rsqrt(var + 1e-5)
    c = _elu((y_ref[...] - mu) * inv * g_ref[...] + be_ref[...])
    c_ref[...] = c
    h = jnp.dot(c, w1_ref[...], preferred_element_type=jnp.float32)
    h_ref[...] = h
    asr = asr_ref[...]
    adr = adr_ref[...]
    as0 = jnp.sum(h[:, :HID] * asr[0], axis=1)
    as1 = jnp.sum(h[:, HID:] * asr[1], axis=1)
    ad0 = jnp.sum(h[:, :HID] * adr[0], axis=1)
    ad1 = jnp.sum(h[:, HID:] * adr[1], axis=1)
    aa_ref[...] = jnp.stack([as0, as1, ad0, ad1], axis=1)
    mblk = jnp.full((8, 128), jnp.maximum(jnp.max(as0), jnp.max(as1)),
                    jnp.float32)

    @pl.when(i == 0)
    def _():
        m_ref[...] = mblk

    @pl.when(i != 0)
    def _():
        m_ref[...] = jnp.maximum(m_ref[...], mblk)


def _tc2(y, stats, gamma, beta, w1, a_src, a_dst):
    return pl.pallas_call(
        _tc2_body,
        grid=(N // BLK,),
        in_specs=[
            pl.BlockSpec((BLK, HID), lambda i: (i, 0)),
            pl.BlockSpec((8, HID), lambda i: (0, 0)),
            pl.BlockSpec((HID,), lambda i: (0,)),
            pl.BlockSpec((HID,), lambda i: (0,)),
            pl.BlockSpec((HID, D), lambda i: (0, 0)),
            pl.BlockSpec((HEADS, HID), lambda i: (0, 0)),
            pl.BlockSpec((HEADS, HID), lambda i: (0, 0)),
        ],
        out_specs=[
            pl.BlockSpec((BLK, HID), lambda i: (i, 0)),
            pl.BlockSpec((BLK, D), lambda i: (i, 0)),
            pl.BlockSpec((BLK, 4), lambda i: (i, 0)),
            pl.BlockSpec((8, 128), lambda i: (0, 0)),
        ],
        out_shape=[
            jax.ShapeDtypeStruct((N, HID), jnp.float32),
            jax.ShapeDtypeStruct((N, D), jnp.float32),
            jax.ShapeDtypeStruct((N, 4), jnp.float32),
            jax.ShapeDtypeStruct((8, 128), jnp.float32),
        ],
    )(y, stats, gamma, beta, w1, a_src, a_dst)


def _gat_post(nd_ref, bias_ref):
    nd = nd_ref[...]
    h0 = nd[0, :, :HID] / nd[0, :, HID:HID + 1]
    h1 = nd[1, :, :HID] / nd[1, :, HID:HID + 1]
    return _elu(jnp.concatenate([h0, h1], axis=1) + bias_ref[...])


def _tc3_body(nd_ref, b1_ref, c_ref, w2_ref, asr_ref, adr_ref,
              cb_ref, h_ref, aa_ref, m_ref):
    i = pl.program_id(0)
    h_i = _gat_post(nd_ref, b1_ref)
    c = c_ref[...]
    cexp = jnp.concatenate([c, c], axis=1)
    cb = cexp - h_i
    cb_ref[...] = cb
    h = jnp.dot(cb, w2_ref[...], preferred_element_type=jnp.float32)
    h_ref[...] = h
    asr = asr_ref[...]
    adr = adr_ref[...]
    as0 = jnp.sum(h[:, :HID] * asr[0], axis=1)
    as1 = jnp.sum(h[:, HID:] * asr[1], axis=1)
    ad0 = jnp.sum(h[:, :HID] * adr[0], axis=1)
    ad1 = jnp.sum(h[:, HID:] * adr[1], axis=1)
    aa_ref[...] = jnp.stack([as0, as1, ad0, ad1], axis=1)
    mblk = jnp.full((8, 128), jnp.maximum(jnp.max(as0), jnp.max(as1)),
                    jnp.float32)

    @pl.when(i == 0)
    def _():
        m_ref[...] = mblk

    @pl.when(i != 0)
    def _():
        m_ref[...] = jnp.maximum(m_ref[...], mblk)


def _tc3(nd, bias1, c, w2, a_src, a_dst):
    return pl.pallas_call(
        _tc3_body,
        grid=(N // BLK,),
        in_specs=[
            pl.BlockSpec((2, BLK, HID + 16), lambda i: (0, i, 0)),
            pl.BlockSpec((D,), lambda i: (0,)),
            pl.BlockSpec((BLK, HID), lambda i: (i, 0)),
            pl.BlockSpec((D, D), lambda i: (0, 0)),
            pl.BlockSpec((HEADS, HID), lambda i: (0, 0)),
            pl.BlockSpec((HEADS, HID), lambda i: (0, 0)),
        ],
        out_specs=[
            pl.BlockSpec((BLK, D), lambda i: (i, 0)),
            pl.BlockSpec((BLK, D), lambda i: (i, 0)),
            pl.BlockSpec((BLK, 4), lambda i: (i, 0)),
            pl.BlockSpec((8, 128), lambda i: (0, 0)),
        ],
        out_shape=[
            jax.ShapeDtypeStruct((N, D), jnp.float32),
            jax.ShapeDtypeStruct((N, D), jnp.float32),
            jax.ShapeDtypeStruct((N, 4), jnp.float32),
            jax.ShapeDtypeStruct((8, 128), jnp.float32),
        ],
    )(nd, bias1, c, w2, a_src, a_dst)


def _tc4_body(nd_ref, b2_ref, c_ref, cb_ref, wf1_ref, bf1_ref,
              wf2_ref, bf2_ref, df_ref):
    h_u = _gat_post(nd_ref, b2_ref)
    cb = cb_ref[...]
    cbu = cb - h_u
    c = c_ref[...]
    cexp = jnp.concatenate([c, c], axis=1)
    wf1 = wf1_ref[...]
    z = (jnp.dot(cexp, wf1[:D], preferred_element_type=jnp.float32)
         + jnp.dot(cb, wf1[D:2 * D], preferred_element_type=jnp.float32)
         + jnp.dot(cbu, wf1[2 * D:], preferred_element_type=jnp.float32)
         + bf1_ref[...])
    z = _lrelu(z)
    df = jnp.sum(z * wf2_ref[...], axis=1) + bf2_ref[0]
    df_ref[...] = jnp.broadcast_to(df[:, None], (BLK, 8))


def _tc4(nd, bias2, c, cb, wf1, bf1, wf2r, bf2):
    return pl.pallas_call(
        _tc4_body,
        grid=(N // BLK,),
        in_specs=[
            pl.BlockSpec((2, BLK, HID + 16), lambda i: (0, i, 0)),
            pl.BlockSpec((D,), lambda i: (0,)),
            pl.BlockSpec((BLK, HID), lambda i: (i, 0)),
            pl.BlockSpec((BLK, D), lambda i: (i, 0)),
            pl.BlockSpec((3 * D, HID), lambda i: (0, 0)),
            pl.BlockSpec((HID,), lambda i: (0,)),
            pl.BlockSpec((1, HID), lambda i: (0, 0)),
            pl.BlockSpec((1,), lambda i: (0,)),
        ],
        out_specs=[pl.BlockSpec((BLK, 8), lambda i: (i, 0))],
        out_shape=[jax.ShapeDtypeStruct((N, 8), jnp.float32)],
    )(nd, bias2, c, cb, wf1, bf1, wf2r, bf2)


def _sc_gat(h_pad, aa2, m2, src3, dst3, nch):
    mesh = plsc.VectorSubcoreMesh(core_axis_name="c", subcore_axis_name="s",
                                  num_cores=NC, num_subcores=NS)

    W = HID + 16

    @functools.partial(
        pl.kernel,
        out_type=jax.ShapeDtypeStruct((NC, NP, W), jnp.float32),
        mesh=mesh,
        compiler_params=pltpu.CompilerParams(needs_layout_passes=False,
                                             use_tc_tiling_on_sc=False),
        scratch_types=[
            pltpu.VMEM((SB, CH), jnp.int32),
            pltpu.VMEM((SB, CH), jnp.int32),
            pltpu.VMEM((NP * 2,), jnp.float32),
            pltpu.VMEM((16,), jnp.float32),
            pltpu.VMEM((2, CH, HID), jnp.float32),
            pltpu.VMEM((2, CH, W), jnp.float32),
            pltpu.VMEM_SHARED((NP, W), jnp.float32),
            pltpu.SemaphoreType.DMA,
            pltpu.SemaphoreType.DMA,
            pltpu.SemaphoreType.DMA,
            pltpu.SemaphoreType.DMA,
        ],
    )
    def k(h_hbm, aa_hbm, m_hbm, src_hbm, dst_hbm, out_hbm,
          srcv, dstv, aav, mv, rowsv, stgv, acc_sh, gsem0, gsem1,
          ssem0, ssem1):
        cid = lax.axis_index("c")
        sid = lax.axis_index("s")

        def zero_rows(r, _):
            for cb in range(W // L):
                stgv[0, r, pl.ds(cb * L, L)] = jnp.zeros((L,), jnp.float32)
                stgv[1, r, pl.ds(cb * L, L)] = jnp.zeros((L,), jnp.float32)
            return 0

        lax.fori_loop(0, CH, zero_rows, 0)

        base = sid * ROWS_PT
        for kk in range(ROWS_PT // CH):
            pltpu.sync_copy(stgv.at[0], acc_sh.at[pl.ds(base + kk * CH, CH), :])

        pltpu.sync_copy(aa_hbm.at[cid], aav)
        pltpu.sync_copy(m_hbm.at[cid], mv)
        m0 = mv[...][0]
        oh = (lax.iota(jnp.int32, L) == 0).astype(jnp.float32)

        plsc.subcore_barrier()

        def start_gather(ci, b):
            pltpu.async_copy(h_hbm.at[cid].at[srcv.at[ci]], rowsv.at[b],
                             gsem0 if b == 0 else gsem1)

        def wait_gather(b):
            pltpu.make_async_copy(
                h_hbm.at[cid].at[srcv.at[0]], rowsv.at[b],
                gsem0 if b == 0 else gsem1).wait()

        def wait_scatter(b):
            pltpu.make_async_copy(
                stgv.at[b], acc_sh.at[dstv.at[0]],
                ssem0 if b == 0 else ssem1).wait()

        def process(ci, b):
            for g in range(CH // L):
                sg2 = srcv[ci, pl.ds(g * L, L)] * 2
                dg2 = dstv[ci, pl.ds(g * L, L)] * 2
                asv = plsc.load_gather(aav, [sg2])
                adv = plsc.load_gather(aav, [dg2 + 1])
                w = jnp.exp(_lrelu(asv + adv) - _lrelu(m0 + adv))
                for j in range(L):
                    e = g * L + j
                    for rb in range(HID // L):
                        stgv[b, e, pl.ds(rb * L, L)] = (
                            rowsv[b, e, pl.ds(rb * L, L)] * w[j])
                    stgv[b, e, pl.ds(HID, L)] = oh * w[j]

            pltpu.async_copy(stgv.at[b], acc_sh.at[dstv.at[ci]],
                             ssem0 if b == 0 else ssem1, add=True)

        def superchunk(sci, _):
            pltpu.sync_copy(src_hbm.at[sid, pl.ds(sci * SB, SB)], srcv)
            pltpu.sync_copy(dst_hbm.at[sid, pl.ds(sci * SB, SB)], dstv)
            start_gather(0, 0)

            def pair(cj, _):
                ci0 = 2 * cj
                start_gather(ci0 + 1, 1)
                wait_gather(0)

                @pl.when((sci > 0) | (cj > 0))
                def _():
                    wait_scatter(0)

                process(ci0, 0)

                @pl.when(ci0 + 2 < SB)
                def _():
                    start_gather(ci0 + 2, 0)

                wait_gather(1)

                @pl.when((sci > 0) | (cj > 0))
                def _():
                    wait_scatter(1)

                process(ci0 + 1, 1)
                return 0

            lax.fori_loop(0, SB // 2, pair, 0)
            return 0

        lax.fori_loop(0, nch // SB, superchunk, 0)
        wait_scatter(0)
        wait_scatter(1)

        plsc.subcore_barrier()

        for kk in range(ROWS_PT // CH):
            r0 = base + kk * CH
            pltpu.sync_copy(acc_sh.at[pl.ds(r0, CH), :], stgv.at[0])
            pltpu.sync_copy(stgv.at[0], out_hbm.at[cid, pl.ds(r0, CH), :])

    return k(h_pad, aa2, m2, src3, dst3)


def _prep_edges(ei):
    e = ei.shape[1] + N
    nch = -(-e // (NS * CH))
    nch = -(-nch // SB) * SB
    epad = NS * CH * nch
    loop = jnp.arange(N, dtype=jnp.int32)
    src = jnp.concatenate(
        [ei[0].astype(jnp.int32), loop,
         jnp.full((epad - e,), NP - 1, jnp.int32)])
    dst = jnp.concatenate(
        [ei[1].astype(jnp.int32), loop,
         jnp.full((epad - e,), NP - 1, jnp.int32)])
    return src.reshape(NS, nch, CH), dst.reshape(NS, nch, CH), nch


def kernel(x_raw, edge_industry, edge_universe, W_enc, b_enc, gamma, beta,
           W1, att_src1, att_dst1, bias1, W2, att_src2, att_dst2, bias2,
           Wf1, bf1, Wf2, bf2, Wa, ba):
    y, stats, attn_weights, fe8 = _tc1(x_raw, W_enc, b_enc, Wa, ba)
    c, h1, aa1, m1 = _tc2(y, stats, gamma, beta, W1, att_src1, att_dst1)

    src1, dst1, nch1 = _prep_edges(edge_industry)
    src2, dst2, nch2 = _prep_edges(edge_universe)

    def _sc_inputs(h, aa, m):
        hp = jnp.pad(h, ((0, NP - N), (0, 0)))
        hs = jnp.stack([hp[:, :HID], hp[:, HID:]])
        aap = jnp.pad(aa, ((0, NP - N), (0, 0)))
        aa2c = jnp.stack(
            [jnp.stack([aap[:, hd], aap[:, 2 + hd]], axis=-1).reshape(-1)
             for hd in range(HEADS)])
        m2c = jnp.broadcast_to(m[0, :16][None], (2, 16))
        return hs, aa2c, m2c

    h1p, aat1, m1c = _sc_inputs(h1, aa1, m1)
    nd1 = _sc_gat(h1p, aat1, m1c, src1, dst1, nch1)

    cb, h2, aa2, m2 = _tc3(nd1[:, :N], bias1, c, W2, att_src2, att_dst2)

    h2p, aat2, m2c = _sc_inputs(h2, aa2, m2)
    nd2 = _sc_gat(h2p, aat2, m2c, src2, dst2, nch2)

    df8 = _tc4(nd2[:, :N], bias2, c, cb, Wf1, bf1,
               Wf2.reshape(1, HID), bf2)[0]

    deep_factor = df8[:, 0]
    factor_estimate = fe8[:, 0]
    return (deep_factor, factor_estimate, attn_weights)


# speedup vs baseline: 39.1951x; 1.0223x over previous
import functools

import jax
import jax.numpy as jnp
from jax import lax
from jax.experimental import pallas as pl
from jax.experimental.pallas import tpu as pltpu
from jax.experimental.pallas import tpu_sc as plsc

N = 10000
F_IN = 256
HID = 64
HEADS = 2
D = HEADS * HID

NC = 2
NS = 16
NW = NC * NS
L = 16

NP = 10240
ROWS_PT = NP // NS
CH = 128
SB = 8
BLK = 1000


def _elu(x):
    return jnp.where(x > 0, x, jnp.exp(x) - 1.0)


def _lrelu(x):
    return jnp.maximum(x, 0.2 * x)


def _tc1_body(x_ref, w_ref, b_ref, wa_ref, ba_ref,
              y_ref, stats_ref, aw_ref, fe_ref):
    i = pl.program_id(0)
    x = x_ref[...]
    y = jnp.dot(x, w_ref[...], preferred_element_type=jnp.float32) + b_ref[...]
    y_ref[...] = y
    s = jnp.sum(y, axis=0)
    sq = jnp.sum(y * y, axis=0)
    st = jnp.concatenate([s[None], sq[None], jnp.zeros((6, HID), jnp.float32)], 0)

    @pl.when(i == 0)
    def _():
        stats_ref[...] = st

    @pl.when(i != 0)
    def _():
        stats_ref[...] += st

    logits = jnp.dot(x, wa_ref[...], preferred_element_type=jnp.float32) + ba_ref[...]
    m = jnp.max(logits, axis=1, keepdims=True)
    e = jnp.exp(logits - m)
    w = e / jnp.sum(e, axis=1, keepdims=True)
    aw_ref[...] = w
    fe = jnp.sum(x * w, axis=1)
    fe_ref[...] = jnp.broadcast_to(fe[:, None], (BLK, 8))


def _tc1(x, w_enc, b_enc, wa, ba):
    return pl.pallas_call(
        _tc1_body,
        grid=(N // BLK,),
        in_specs=[
            pl.BlockSpec((BLK, F_IN), lambda i: (i, 0)),
            pl.BlockSpec((F_IN, HID), lambda i: (0, 0)),
            pl.BlockSpec((HID,), lambda i: (0,)),
            pl.BlockSpec((F_IN, F_IN), lambda i: (0, 0)),
            pl.BlockSpec((F_IN,), lambda i: (0,)),
        ],
        out_specs=[
            pl.BlockSpec((BLK, HID), lambda i: (i, 0)),
            pl.BlockSpec((8, HID), lambda i: (0, 0)),
            pl.BlockSpec((BLK, F_IN), lambda i: (i, 0)),
            pl.BlockSpec((BLK, 8), lambda i: (i, 0)),
        ],
        out_shape=[
            jax.ShapeDtypeStruct((N, HID), jnp.float32),
            jax.ShapeDtypeStruct((8, HID), jnp.float32),
            jax.ShapeDtypeStruct((N, F_IN), jnp.float32),
            jax.ShapeDtypeStruct((N, 8), jnp.float32),
        ],
    )(x, w_enc, b_enc, wa, ba)


def _tc2_body(y_ref, stats_ref, g_ref, be_ref, w1_ref, asr_ref, adr_ref,
              c_ref, h_ref, aa_ref, m_ref):
    i = pl.program_id(0)
    st = stats_ref[...]
    mu = st[0] / N
    var = st[1] / N - mu * mu
    inv = lax.rsqrt(var + 1e-5)
    c = _elu((y_ref[...] - mu) * inv * g_ref[...] + be_ref[...])
    c_ref[...] = c
    h = jnp.dot(c, w1_ref[...], preferred_element_type=jnp.float32)
    h_ref[...] = h
    asr = asr_ref[...]
    adr = adr_ref[...]
    as0 = jnp.sum(h[:, :HID] * asr[0], axis=1)
    as1 = jnp.sum(h[:, HID:] * asr[1], axis=1)
    ad0 = jnp.sum(h[:, :HID] * adr[0], axis=1)
    ad1 = jnp.sum(h[:, HID:] * adr[1], axis=1)
    aa_ref[...] = jnp.stack([as0, as1, ad0, ad1], axis=1)
    mblk = jnp.full((8, 128), jnp.maximum(jnp.max(as0), jnp.max(as1)),
                    jnp.float32)

    @pl.when(i == 0)
    def _():
        m_ref[...] = mblk

    @pl.when(i != 0)
    def _():
        m_ref[...] = jnp.maximum(m_ref[...], mblk)


def _tc2(y, stats, gamma, beta, w1, a_src, a_dst):
    return pl.pallas_call(
        _tc2_body,
        grid=(N // BLK,),
        in_specs=[
            pl.BlockSpec((BLK, HID), lambda i: (i, 0)),
            pl.BlockSpec((8, HID), lambda i: (0, 0)),
            pl.BlockSpec((HID,), lambda i: (0,)),
            pl.BlockSpec((HID,), lambda i: (0,)),
            pl.BlockSpec((HID, D), lambda i: (0, 0)),
            pl.BlockSpec((HEADS, HID), lambda i: (0, 0)),
            pl.BlockSpec((HEADS, HID), lambda i: (0, 0)),
        ],
        out_specs=[
            pl.BlockSpec((BLK, HID), lambda i: (i, 0)),
            pl.BlockSpec((BLK, D), lambda i: (i, 0)),
            pl.BlockSpec((BLK, 4), lambda i: (i, 0)),
            pl.BlockSpec((8, 128), lambda i: (0, 0)),
        ],
        out_shape=[
            jax.ShapeDtypeStruct((N, HID), jnp.float32),
            jax.ShapeDtypeStruct((N, D), jnp.float32),
            jax.ShapeDtypeStruct((N, 4), jnp.float32),
            jax.ShapeDtypeStruct((8, 128), jnp.float32),
        ],
    )(y, stats, gamma, beta, w1, a_src, a_dst)


def _gat_post(nd_ref, bias_ref):
    nd = nd_ref[...]
    h0 = nd[0, :, :HID] / nd[0, :, HID:HID + 1]
    h1 = nd[1, :, :HID] / nd[1, :, HID:HID + 1]
    return _elu(jnp.concatenate([h0, h1], axis=1) + bias_ref[...])


def _tc3_body(nd_ref, b1_ref, c_ref, w2_ref, asr_ref, adr_ref,
              cb_ref, h_ref, aa_ref, m_ref):
    i = pl.program_id(0)
    h_i = _gat_post(nd_ref, b1_ref)
    c = c_ref[...]
    cexp = jnp.concatenate([c, c], axis=1)
    cb = cexp - h_i
    cb_ref[...] = cb
    h = jnp.dot(cb, w2_ref[...], preferred_element_type=jnp.float32)
    h_ref[...] = h
    asr = asr_ref[...]
    adr = adr_ref[...]
    as0 = jnp.sum(h[:, :HID] * asr[0], axis=1)
    as1 = jnp.sum(h[:, HID:] * asr[1], axis=1)
    ad0 = jnp.sum(h[:, :HID] * adr[0], axis=1)
    ad1 = jnp.sum(h[:, HID:] * adr[1], axis=1)
    aa_ref[...] = jnp.stack([as0, as1, ad0, ad1], axis=1)
    mblk = jnp.full((8, 128), jnp.maximum(jnp.max(as0), jnp.max(as1)),
                    jnp.float32)

    @pl.when(i == 0)
    def _():
        m_ref[...] = mblk

    @pl.when(i != 0)
    def _():
        m_ref[...] = jnp.maximum(m_ref[...], mblk)


def _tc3(nd, bias1, c, w2, a_src, a_dst):
    return pl.pallas_call(
        _tc3_body,
        grid=(N // BLK,),
        in_specs=[
            pl.BlockSpec((2, BLK, HID + 16), lambda i: (0, i, 0)),
            pl.BlockSpec((D,), lambda i: (0,)),
            pl.BlockSpec((BLK, HID), lambda i: (i, 0)),
            pl.BlockSpec((D, D), lambda i: (0, 0)),
            pl.BlockSpec((HEADS, HID), lambda i: (0, 0)),
            pl.BlockSpec((HEADS, HID), lambda i: (0, 0)),
        ],
        out_specs=[
            pl.BlockSpec((BLK, D), lambda i: (i, 0)),
            pl.BlockSpec((BLK, D), lambda i: (i, 0)),
            pl.BlockSpec((BLK, 4), lambda i: (i, 0)),
            pl.BlockSpec((8, 128), lambda i: (0, 0)),
        ],
        out_shape=[
            jax.ShapeDtypeStruct((N, D), jnp.float32),
            jax.ShapeDtypeStruct((N, D), jnp.float32),
            jax.ShapeDtypeStruct((N, 4), jnp.float32),
            jax.ShapeDtypeStruct((8, 128), jnp.float32),
        ],
    )(nd, bias1, c, w2, a_src, a_dst)


def _tc4_body(nd_ref, b2_ref, c_ref, cb_ref, wf1_ref, bf1_ref,
              wf2_ref, bf2_ref, df_ref):
    h_u = _gat_post(nd_ref, b2_ref)
    cb = cb_ref[...]
    cbu = cb - h_u
    c = c_ref[...]
    cexp = jnp.concatenate([c, c], axis=1)
    wf1 = wf1_ref[...]
    z = (jnp.dot(cexp, wf1[:D], preferred_element_type=jnp.float32)
         + jnp.dot(cb, wf1[D:2 * D], preferred_element_type=jnp.float32)
         + jnp.dot(cbu, wf1[2 * D:], preferred_element_type=jnp.float32)
         + bf1_ref[...])
    z = _lrelu(z)
    df = jnp.sum(z * wf2_ref[...], axis=1) + bf2_ref[0]
    df_ref[...] = jnp.broadcast_to(df[:, None], (BLK, 8))


def _tc4(nd, bias2, c, cb, wf1, bf1, wf2r, bf2):
    return pl.pallas_call(
        _tc4_body,
        grid=(N // BLK,),
        in_specs=[
            pl.BlockSpec((2, BLK, HID + 16), lambda i: (0, i, 0)),
            pl.BlockSpec((D,), lambda i: (0,)),
            pl.BlockSpec((BLK, HID), lambda i: (i, 0)),
            pl.BlockSpec((BLK, D), lambda i: (i, 0)),
            pl.BlockSpec((3 * D, HID), lambda i: (0, 0)),
            pl.BlockSpec((HID,), lambda i: (0,)),
            pl.BlockSpec((1, HID), lambda i: (0, 0)),
            pl.BlockSpec((1,), lambda i: (0,)),
        ],
        out_specs=[pl.BlockSpec((BLK, 8), lambda i: (i, 0))],
        out_shape=[jax.ShapeDtypeStruct((N, 8), jnp.float32)],
    )(nd, bias2, c, cb, wf1, bf1, wf2r, bf2)


def _sc_gat(h_pad, aa2, m2, src3, dst3, nch):
    mesh = plsc.VectorSubcoreMesh(core_axis_name="c", subcore_axis_name="s",
                                  num_cores=NC, num_subcores=NS)

    W = HID + 16

    @functools.partial(
        pl.kernel,
        out_type=jax.ShapeDtypeStruct((NC, NP, W), jnp.float32),
        mesh=mesh,
        compiler_params=pltpu.CompilerParams(needs_layout_passes=False,
                                             use_tc_tiling_on_sc=False),
        scratch_types=[
            pltpu.VMEM((SB, CH), jnp.int32),
            pltpu.VMEM((SB, CH), jnp.int32),
            pltpu.VMEM((NP * 2,), jnp.float32),
            pltpu.VMEM((16,), jnp.float32),
            pltpu.VMEM((2, CH, HID), jnp.float32),
            pltpu.VMEM((2, CH, W), jnp.float32),
            pltpu.VMEM_SHARED((NP, W), jnp.float32),
            pltpu.SemaphoreType.DMA,
            pltpu.SemaphoreType.DMA,
            pltpu.SemaphoreType.DMA,
            pltpu.SemaphoreType.DMA,
        ],
    )
    def k(h_hbm, aa_hbm, m_hbm, src_hbm, dst_hbm, out_hbm,
          srcv, dstv, aav, mv, rowsv, stgv, acc_sh, gsem0, gsem1,
          ssem0, ssem1):
        cid = lax.axis_index("c")
        sid = lax.axis_index("s")

        def zero_rows(r, _):
            for cb in range(W // L):
                stgv[0, r, pl.ds(cb * L, L)] = jnp.zeros((L,), jnp.float32)
                stgv[1, r, pl.ds(cb * L, L)] = jnp.zeros((L,), jnp.float32)
            return 0

        lax.fori_loop(0, CH, zero_rows, 0)

        base = sid * ROWS_PT
        for kk in range(ROWS_PT // CH):
            pltpu.sync_copy(stgv.at[0], acc_sh.at[pl.ds(base + kk * CH, CH), :])

        pltpu.sync_copy(aa_hbm.at[cid], aav)
        pltpu.sync_copy(m_hbm.at[cid], mv)
        m0 = mv[...][0]
        zi = jnp.zeros((L,), jnp.int32)

        plsc.subcore_barrier()

        def start_gather(ci, b):
            pltpu.async_copy(h_hbm.at[cid].at[srcv.at[ci]], rowsv.at[b],
                             gsem0 if b == 0 else gsem1)

        def wait_gather(b):
            pltpu.make_async_copy(
                h_hbm.at[cid].at[srcv.at[0]], rowsv.at[b],
                gsem0 if b == 0 else gsem1).wait()

        def wait_scatter(b):
            pltpu.make_async_copy(
                stgv.at[b], acc_sh.at[dstv.at[0]],
                ssem0 if b == 0 else ssem1).wait()

        def process(ci, b):
            for g in range(CH // L):
                sg2 = srcv[ci, pl.ds(g * L, L)] * 2
                dg2 = dstv[ci, pl.ds(g * L, L)] * 2
                asv = plsc.load_gather(aav, [sg2])
                adv = plsc.load_gather(aav, [dg2 + 1])
                w = jnp.exp(_lrelu(asv + adv) - _lrelu(m0 + adv))
                rows16 = lax.iota(jnp.int32, L) + g * L
                plsc.store_scatter(stgv.at[b], [rows16, zi + HID], w)
                for j in range(L):
                    e = g * L + j
                    for rb in range(HID // L):
                        stgv[b, e, pl.ds(rb * L, L)] = (
                            rowsv[b, e, pl.ds(rb * L, L)] * w[j])

            pltpu.async_copy(stgv.at[b], acc_sh.at[dstv.at[ci]],
                             ssem0 if b == 0 else ssem1, add=True)

        def superchunk(sci, _):
            pltpu.sync_copy(src_hbm.at[sid, pl.ds(sci * SB, SB)], srcv)
            pltpu.sync_copy(dst_hbm.at[sid, pl.ds(sci * SB, SB)], dstv)
            start_gather(0, 0)

            def pair(cj, _):
                ci0 = 2 * cj
                start_gather(ci0 + 1, 1)
                wait_gather(0)

                @pl.when((sci > 0) | (cj > 0))
                def _():
                    wait_scatter(0)

                process(ci0, 0)

                @pl.when(ci0 + 2 < SB)
                def _():
                    start_gather(ci0 + 2, 0)

                wait_gather(1)

                @pl.when((sci > 0) | (cj > 0))
                def _():
                    wait_scatter(1)

                process(ci0 + 1, 1)
                return 0

            lax.fori_loop(0, SB // 2, pair, 0)
            return 0

        lax.fori_loop(0, nch // SB, superchunk, 0)
        wait_scatter(0)
        wait_scatter(1)

        plsc.subcore_barrier()

        for kk in range(ROWS_PT // CH):
            r0 = base + kk * CH
            pltpu.sync_copy(acc_sh.at[pl.ds(r0, CH), :], stgv.at[0])
            pltpu.sync_copy(stgv.at[0], out_hbm.at[cid, pl.ds(r0, CH), :])

    return k(h_pad, aa2, m2, src3, dst3)


def _prep_edges(ei):
    e = ei.shape[1] + N
    nch = -(-e // (NS * CH))
    nch = -(-nch // SB) * SB
    epad = NS * CH * nch
    loop = jnp.arange(N, dtype=jnp.int32)
    src = jnp.concatenate(
        [ei[0].astype(jnp.int32), loop,
         jnp.full((epad - e,), NP - 1, jnp.int32)])
    dst = jnp.concatenate(
        [ei[1].astype(jnp.int32), loop,
         jnp.full((epad - e,), NP - 1, jnp.int32)])
    return src.reshape(NS, nch, CH), dst.reshape(NS, nch, CH), nch


def kernel(x_raw, edge_industry, edge_universe, W_enc, b_enc, gamma, beta,
           W1, att_src1, att_dst1, bias1, W2, att_src2, att_dst2, bias2,
           Wf1, bf1, Wf2, bf2, Wa, ba):
    y, stats, attn_weights, fe8 = _tc1(x_raw, W_enc, b_enc, Wa, ba)
    c, h1, aa1, m1 = _tc2(y, stats, gamma, beta, W1, att_src1, att_dst1)

    src1, dst1, nch1 = _prep_edges(edge_industry)
    src2, dst2, nch2 = _prep_edges(edge_universe)

    def _sc_inputs(h, aa, m):
        hp = jnp.pad(h, ((0, NP - N), (0, 0)))
        hs = jnp.stack([hp[:, :HID], hp[:, HID:]])
        aap = jnp.pad(aa, ((0, NP - N), (0, 0)))
        aa2c = jnp.stack(
            [jnp.stack([aap[:, hd], aap[:, 2 + hd]], axis=-1).reshape(-1)
             for hd in range(HEADS)])
        m2c = jnp.broadcast_to(m[0, :16][None], (2, 16))
        return hs, aa2c, m2c

    h1p, aat1, m1c = _sc_inputs(h1, aa1, m1)
    nd1 = _sc_gat(h1p, aat1, m1c, src1, dst1, nch1)

    cb, h2, aa2, m2 = _tc3(nd1[:, :N], bias1, c, W2, att_src2, att_dst2)

    h2p, aat2, m2c = _sc_inputs(h2, aa2, m2)
    nd2 = _sc_gat(h2p, aat2, m2c, src2, dst2, nch2)

    df8 = _tc4(nd2[:, :N], bias2, c, cb, Wf1, bf1,
               Wf2.reshape(1, HID), bf2)[0]

    deep_factor = df8[:, 0]
    factor_estimate = fe8[:, 0]
    return (deep_factor, factor_estimate, attn_weights)


# speedup vs baseline: 39.2150x; 1.0005x over previous
import functools

import jax
import jax.numpy as jnp
from jax import lax
from jax.experimental import pallas as pl
from jax.experimental.pallas import tpu as pltpu
from jax.experimental.pallas import tpu_sc as plsc

N = 10000
F_IN = 256
HID = 64
HEADS = 2
D = HEADS * HID

NC = 2
NS = 16
L = 16

NP = 10240
ROWS_PT = NP // NS
CH = 128
SB = 8
BLK = 1000


def _elu(x):
    return jnp.where(x > 0, x, jnp.exp(x) - 1.0)


def _lrelu(x):
    return jnp.maximum(x, 0.2 * x)


def _tc1_body(x_ref, w_ref, b_ref, wa_ref, ba_ref,
              y_ref, stats_ref, aw_ref, fe_ref):
    i = pl.program_id(0)
    x = x_ref[...]
    y = jnp.dot(x, w_ref[...], preferred_element_type=jnp.float32) + b_ref[...]
    y_ref[...] = y
    s = jnp.sum(y, axis=0)
    sq = jnp.sum(y * y, axis=0)
    st = jnp.concatenate([s[None], sq[None], jnp.zeros((6, HID), jnp.float32)], 0)

    @pl.when(i == 0)
    def _():
        stats_ref[...] = st

    @pl.when(i != 0)
    def _():
        stats_ref[...] += st

    logits = jnp.dot(x, wa_ref[...], preferred_element_type=jnp.float32) + ba_ref[...]
    m = jnp.max(logits, axis=1, keepdims=True)
    e = jnp.exp(logits - m)
    w = e / jnp.sum(e, axis=1, keepdims=True)
    aw_ref[...] = w
    fe = jnp.sum(x * w, axis=1)
    fe_ref[...] = jnp.broadcast_to(fe[:, None], (BLK, 8))


def _tc1(x, w_enc, b_enc, wa, ba):
    return pl.pallas_call(
        _tc1_body,
        grid=(N // BLK,),
        in_specs=[
            pl.BlockSpec((BLK, F_IN), lambda i: (i, 0)),
            pl.BlockSpec((F_IN, HID), lambda i: (0, 0)),
            pl.BlockSpec((HID,), lambda i: (0,)),
            pl.BlockSpec((F_IN, F_IN), lambda i: (0, 0)),
            pl.BlockSpec((F_IN,), lambda i: (0,)),
        ],
        out_specs=[
            pl.BlockSpec((BLK, HID), lambda i: (i, 0)),
            pl.BlockSpec((8, HID), lambda i: (0, 0)),
            pl.BlockSpec((BLK, F_IN), lambda i: (i, 0)),
            pl.BlockSpec((BLK, 8), lambda i: (i, 0)),
        ],
        out_shape=[
            jax.ShapeDtypeStruct((N, HID), jnp.float32),
            jax.ShapeDtypeStruct((8, HID), jnp.float32),
            jax.ShapeDtypeStruct((N, F_IN), jnp.float32),
            jax.ShapeDtypeStruct((N, 8), jnp.float32),
        ],
    )(x, w_enc, b_enc, wa, ba)


def _tc2_body(y_ref, stats_ref, g_ref, be_ref, w1_ref, asr_ref, adr_ref,
              c_ref, h_ref, aa_ref, m_ref):
    i = pl.program_id(0)
    st = stats_ref[...]
    mu = st[0] / N
    var = st[1] / N - mu * mu
    inv = lax.rsqrt(var + 1e-5)
    c = _elu((y_ref[...] - mu) * inv * g_ref[...] + be_ref[...])
    c_ref[...] = c
    h = jnp.dot(c, w1_ref[...], preferred_element_type=jnp.float32)
    h_ref[...] = h
    asr = asr_ref[...]
    adr = adr_ref[...]
    as0 = jnp.sum(h[:, :HID] * asr[0], axis=1)
    as1 = jnp.sum(h[:, HID:] * asr[1], axis=1)
    ad0 = jnp.sum(h[:, :HID] * adr[0], axis=1)
    ad1 = jnp.sum(h[:, HID:] * adr[1], axis=1)
    aa_ref[...] = jnp.stack([as0, as1, ad0, ad1], axis=1)
    mblk = jnp.full((8, 128), jnp.maximum(jnp.max(as0), jnp.max(as1)),
                    jnp.float32)

    @pl.when(i == 0)
    def _():
        m_ref[...] = mblk

    @pl.when(i != 0)
    def _():
        m_ref[...] = jnp.maximum(m_ref[...], mblk)


def _tc2(y, stats, gamma, beta, w1, a_src, a_dst):
    return pl.pallas_call(
        _tc2_body,
        grid=(N // BLK,),
        in_specs=[
            pl.BlockSpec((BLK, HID), lambda i: (i, 0)),
            pl.BlockSpec((8, HID), lambda i: (0, 0)),
            pl.BlockSpec((HID,), lambda i: (0,)),
            pl.BlockSpec((HID,), lambda i: (0,)),
            pl.BlockSpec((HID, D), lambda i: (0, 0)),
            pl.BlockSpec((HEADS, HID), lambda i: (0, 0)),
            pl.BlockSpec((HEADS, HID), lambda i: (0, 0)),
        ],
        out_specs=[
            pl.BlockSpec((BLK, HID), lambda i: (i, 0)),
            pl.BlockSpec((BLK, D), lambda i: (i, 0)),
            pl.BlockSpec((BLK, 4), lambda i: (i, 0)),
            pl.BlockSpec((8, 128), lambda i: (0, 0)),
        ],
        out_shape=[
            jax.ShapeDtypeStruct((N, HID), jnp.float32),
            jax.ShapeDtypeStruct((N, D), jnp.float32),
            jax.ShapeDtypeStruct((N, 4), jnp.float32),
            jax.ShapeDtypeStruct((8, 128), jnp.float32),
        ],
    )(y, stats, gamma, beta, w1, a_src, a_dst)


def _gat_post(nd_ref, bias_ref):
    nd = nd_ref[...]
    h0 = nd[0, :, :HID] / nd[0, :, HID:HID + 1]
    h1 = nd[1, :, :HID] / nd[1, :, HID:HID + 1]
    return _elu(jnp.concatenate([h0, h1], axis=1) + bias_ref[...])


def _tc3_body(nd_ref, b1_ref, c_ref, w2_ref, asr_ref, adr_ref,
              cb_ref, h_ref, aa_ref, m_ref):
    i = pl.program_id(0)
    h_i = _gat_post(nd_ref, b1_ref)
    c = c_ref[...]
    cexp = jnp.concatenate([c, c], axis=1)
    cb = cexp - h_i
    cb_ref[...] = cb
    h = jnp.dot(cb, w2_ref[...], preferred_element_type=jnp.float32)
    h_ref[...] = h
    asr = asr_ref[...]
    adr = adr_ref[...]
    as0 = jnp.sum(h[:, :HID] * asr[0], axis=1)
    as1 = jnp.sum(h[:, HID:] * asr[1], axis=1)
    ad0 = jnp.sum(h[:, :HID] * adr[0], axis=1)
    ad1 = jnp.sum(h[:, HID:] * adr[1], axis=1)
    aa_ref[...] = jnp.stack([as0, as1, ad0, ad1], axis=1)
    mblk = jnp.full((8, 128), jnp.maximum(jnp.max(as0), jnp.max(as1)),
                    jnp.float32)

    @pl.when(i == 0)
    def _():
        m_ref[...] = mblk

    @pl.when(i != 0)
    def _():
        m_ref[...] = jnp.maximum(m_ref[...], mblk)


def _tc3(nd, bias1, c, w2, a_src, a_dst):
    return pl.pallas_call(
        _tc3_body,
        grid=(N // BLK,),
        in_specs=[
            pl.BlockSpec((2, BLK, HID + 16), lambda i: (0, i, 0)),
            pl.BlockSpec((D,), lambda i: (0,)),
            pl.BlockSpec((BLK, HID), lambda i: (i, 0)),
            pl.BlockSpec((D, D), lambda i: (0, 0)),
            pl.BlockSpec((HEADS, HID), lambda i: (0, 0)),
            pl.BlockSpec((HEADS, HID), lambda i: (0, 0)),
        ],
        out_specs=[
            pl.BlockSpec((BLK, D), lambda i: (i, 0)),
            pl.BlockSpec((BLK, D), lambda i: (i, 0)),
            pl.BlockSpec((BLK, 4), lambda i: (i, 0)),
            pl.BlockSpec((8, 128), lambda i: (0, 0)),
        ],
        out_shape=[
            jax.ShapeDtypeStruct((N, D), jnp.float32),
            jax.ShapeDtypeStruct((N, D), jnp.float32),
            jax.ShapeDtypeStruct((N, 4), jnp.float32),
            jax.ShapeDtypeStruct((8, 128), jnp.float32),
        ],
    )(nd, bias1, c, w2, a_src, a_dst)


def _tc4_body(nd_ref, b2_ref, c_ref, cb_ref, wf1_ref, bf1_ref,
              wf2_ref, bf2_ref, df_ref):
    h_u = _gat_post(nd_ref, b2_ref)
    cb = cb_ref[...]
    cbu = cb - h_u
    c = c_ref[...]
    cexp = jnp.concatenate([c, c], axis=1)
    wf1 = wf1_ref[...]
    z = (jnp.dot(cexp, wf1[:D], preferred_element_type=jnp.float32)
         + jnp.dot(cb, wf1[D:2 * D], preferred_element_type=jnp.float32)
         + jnp.dot(cbu, wf1[2 * D:], preferred_element_type=jnp.float32)
         + bf1_ref[...])
    z = _lrelu(z)
    df = jnp.sum(z * wf2_ref[...], axis=1) + bf2_ref[0]
    df_ref[...] = jnp.broadcast_to(df[:, None], (BLK, 8))


def _tc4(nd, bias2, c, cb, wf1, bf1, wf2r, bf2):
    return pl.pallas_call(
        _tc4_body,
        grid=(N // BLK,),
        in_specs=[
            pl.BlockSpec((2, BLK, HID + 16), lambda i: (0, i, 0)),
            pl.BlockSpec((D,), lambda i: (0,)),
            pl.BlockSpec((BLK, HID), lambda i: (i, 0)),
            pl.BlockSpec((BLK, D), lambda i: (i, 0)),
            pl.BlockSpec((3 * D, HID), lambda i: (0, 0)),
            pl.BlockSpec((HID,), lambda i: (0,)),
            pl.BlockSpec((1, HID), lambda i: (0, 0)),
            pl.BlockSpec((1,), lambda i: (0,)),
        ],
        out_specs=[pl.BlockSpec((BLK, 8), lambda i: (i, 0))],
        out_shape=[jax.ShapeDtypeStruct((N, 8), jnp.float32)],
    )(nd, bias2, c, cb, wf1, bf1, wf2r, bf2)


def _sc_gat(h_pad, aa2, m2, src3, dst3, nch):
    mesh = plsc.VectorSubcoreMesh(core_axis_name="c", subcore_axis_name="s",
                                  num_cores=NC, num_subcores=NS)

    W = HID + 16

    @functools.partial(
        pl.kernel,
        out_type=jax.ShapeDtypeStruct((NC, NP, W), jnp.float32),
        mesh=mesh,
        compiler_params=pltpu.CompilerParams(needs_layout_passes=False,
                                             use_tc_tiling_on_sc=False),
        scratch_types=[
            pltpu.VMEM((SB, CH), jnp.int32),
            pltpu.VMEM((SB, CH), jnp.int32),
            pltpu.VMEM((NP * 2,), jnp.float32),
            pltpu.VMEM((16,), jnp.float32),
            pltpu.VMEM((2, CH, HID), jnp.float32),
            pltpu.VMEM((2, CH, W), jnp.float32),
            pltpu.VMEM_SHARED((NP, W), jnp.float32),
            pltpu.SemaphoreType.DMA,
            pltpu.SemaphoreType.DMA,
            pltpu.SemaphoreType.DMA,
            pltpu.SemaphoreType.DMA,
        ],
    )
    def k(h_hbm, aa_hbm, m_hbm, src_hbm, dst_hbm, out_hbm,
          srcv, dstv, aav, mv, rowsv, stgv, acc_sh, gsem0, gsem1,
          ssem0, ssem1):
        cid = lax.axis_index("c")
        sid = lax.axis_index("s")

        def zero_rows(r, _):
            for cb in range(W // L):
                stgv[0, r, pl.ds(cb * L, L)] = jnp.zeros((L,), jnp.float32)
                stgv[1, r, pl.ds(cb * L, L)] = jnp.zeros((L,), jnp.float32)
            return 0

        lax.fori_loop(0, CH, zero_rows, 0)

        base = sid * ROWS_PT
        for kk in range(ROWS_PT // CH):
            pltpu.sync_copy(stgv.at[0], acc_sh.at[pl.ds(base + kk * CH, CH), :])

        pltpu.sync_copy(aa_hbm.at[cid], aav)
        pltpu.sync_copy(m_hbm.at[cid], mv)
        m0 = mv[...][0]
        zi = jnp.zeros((L,), jnp.int32)

        plsc.subcore_barrier()

        def start_gather(ci, b):
            pltpu.async_copy(h_hbm.at[cid].at[srcv.at[ci]], rowsv.at[b],
                             gsem0 if b == 0 else gsem1)

        def wait_gather(b):
            pltpu.make_async_copy(
                h_hbm.at[cid].at[srcv.at[0]], rowsv.at[b],
                gsem0 if b == 0 else gsem1).wait()

        def wait_scatter(b):
            pltpu.make_async_copy(
                stgv.at[b], acc_sh.at[dstv.at[0]],
                ssem0 if b == 0 else ssem1).wait()

        def process(ci, b):
            for g in range(CH // L):
                sg2 = srcv[ci, pl.ds(g * L, L)] * 2
                dg2 = dstv[ci, pl.ds(g * L, L)] * 2
                asv = plsc.load_gather(aav, [sg2])
                adv = plsc.load_gather(aav, [dg2 + 1])
                w = jnp.exp(_lrelu(asv + adv) - _lrelu(m0 + adv))
                rows16 = lax.iota(jnp.int32, L) + g * L
                plsc.store_scatter(stgv.at[b], [rows16, zi + HID], w)
                for j in range(L):
                    e = g * L + j
                    for rb in range(HID // L):
                        stgv[b, e, pl.ds(rb * L, L)] = (
                            rowsv[b, e, pl.ds(rb * L, L)] * w[j])

            pltpu.async_copy(stgv.at[b], acc_sh.at[dstv.at[ci]],
                             ssem0 if b == 0 else ssem1, add=True)

        def superchunk(sci, _):
            pltpu.sync_copy(src_hbm.at[sid, pl.ds(sci * SB, SB)], srcv)
            pltpu.sync_copy(dst_hbm.at[sid, pl.ds(sci * SB, SB)], dstv)
            start_gather(0, 0)

            def pair(cj, _):
                ci0 = 2 * cj
                start_gather(ci0 + 1, 1)
                wait_gather(0)

                @pl.when((sci > 0) | (cj > 0))
                def _():
                    wait_scatter(0)

                process(ci0, 0)

                @pl.when(ci0 + 2 < SB)
                def _():
                    start_gather(ci0 + 2, 0)

                wait_gather(1)

                @pl.when((sci > 0) | (cj > 0))
                def _():
                    wait_scatter(1)

                process(ci0 + 1, 1)
                return 0

            lax.fori_loop(0, SB // 2, pair, 0)
            return 0

        lax.fori_loop(0, nch // SB, superchunk, 0)
        wait_scatter(0)
        wait_scatter(1)

        plsc.subcore_barrier()

        for kk in range(ROWS_PT // CH):
            r0 = base + kk * CH
            pltpu.sync_copy(acc_sh.at[pl.ds(r0, CH), :], stgv.at[0])
            pltpu.sync_copy(stgv.at[0], out_hbm.at[cid, pl.ds(r0, CH), :])

    return k(h_pad, aa2, m2, src3, dst3)


def _prep_edges(ei):
    e = ei.shape[1] + N
    nch = -(-e // (NS * CH))
    nch = -(-nch // SB) * SB
    epad = NS * CH * nch
    loop = jnp.arange(N, dtype=jnp.int32)
    src = jnp.concatenate(
        [ei[0].astype(jnp.int32), loop,
         jnp.full((epad - e,), NP - 1, jnp.int32)])
    dst = jnp.concatenate(
        [ei[1].astype(jnp.int32), loop,
         jnp.full((epad - e,), NP - 1, jnp.int32)])
    return src.reshape(NS, nch, CH), dst.reshape(NS, nch, CH), nch


def kernel(x_raw, edge_industry, edge_universe, W_enc, b_enc, gamma, beta,
           W1, att_src1, att_dst1, bias1, W2, att_src2, att_dst2, bias2,
           Wf1, bf1, Wf2, bf2, Wa, ba):
    y, stats, attn_weights, fe8 = _tc1(x_raw, W_enc, b_enc, Wa, ba)
    c, h1, aa1, m1 = _tc2(y, stats, gamma, beta, W1, att_src1, att_dst1)

    src1, dst1, nch1 = _prep_edges(edge_industry)
    src2, dst2, nch2 = _prep_edges(edge_universe)

    def _sc_inputs(h, aa, m):
        hp = jnp.pad(h, ((0, NP - N), (0, 0)))
        hs = jnp.stack([hp[:, :HID], hp[:, HID:]])
        aap = jnp.pad(aa, ((0, NP - N), (0, 0)))
        aa2c = jnp.stack(
            [jnp.stack([aap[:, hd], aap[:, 2 + hd]], axis=-1).reshape(-1)
             for hd in range(HEADS)])
        m2c = jnp.broadcast_to(m[0, :16][None], (2, 16))
        return hs, aa2c, m2c

    h1p, aat1, m1c = _sc_inputs(h1, aa1, m1)
    nd1 = _sc_gat(h1p, aat1, m1c, src1, dst1, nch1)

    cb, h2, aa2, m2 = _tc3(nd1[:, :N], bias1, c, W2, att_src2, att_dst2)

    h2p, aat2, m2c = _sc_inputs(h2, aa2, m2)
    nd2 = _sc_gat(h2p, aat2, m2c, src2, dst2, nch2)

    df8 = _tc4(nd2[:, :N], bias2, c, cb, Wf1, bf1,
               Wf2.reshape(1, HID), bf2)[0]

    deep_factor = df8[:, 0]
    factor_estimate = fe8[:, 0]
    return (deep_factor, factor_estimate, attn_weights)


# speedup vs baseline: 49.2192x; 1.2551x over previous
import functools

import jax
import jax.numpy as jnp
from jax import lax
from jax.experimental import pallas as pl
from jax.experimental.pallas import tpu as pltpu
from jax.experimental.pallas import tpu_sc as plsc

N = 10000
F_IN = 256
HID = 64
HEADS = 2
D = HEADS * HID

NC = 2
NS = 16
L = 16

NP = 10240
ROWS_PT = NP // NS
CH = 128
SB = 28
BLK = 1000


def _elu(x):
    return jnp.where(x > 0, x, jnp.exp(x) - 1.0)


def _lrelu(x):
    return jnp.maximum(x, 0.2 * x)


def _tc1_body(x_ref, w_ref, b_ref, wa_ref, ba_ref,
              y_ref, stats_ref, aw_ref, fe_ref):
    i = pl.program_id(0)
    x = x_ref[...]
    y = jnp.dot(x, w_ref[...], preferred_element_type=jnp.float32) + b_ref[...]
    y_ref[...] = y
    s = jnp.sum(y, axis=0)
    sq = jnp.sum(y * y, axis=0)
    st = jnp.concatenate([s[None], sq[None], jnp.zeros((6, HID), jnp.float32)], 0)

    @pl.when(i == 0)
    def _():
        stats_ref[...] = st

    @pl.when(i != 0)
    def _():
        stats_ref[...] += st

    logits = jnp.dot(x, wa_ref[...], preferred_element_type=jnp.float32) + ba_ref[...]
    m = jnp.max(logits, axis=1, keepdims=True)
    e = jnp.exp(logits - m)
    w = e / jnp.sum(e, axis=1, keepdims=True)
    aw_ref[...] = w
    fe = jnp.sum(x * w, axis=1)
    fe_ref[...] = jnp.broadcast_to(fe[:, None], (BLK, 8))


def _tc1(x, w_enc, b_enc, wa, ba):
    return pl.pallas_call(
        _tc1_body,
        grid=(N // BLK,),
        in_specs=[
            pl.BlockSpec((BLK, F_IN), lambda i: (i, 0)),
            pl.BlockSpec((F_IN, HID), lambda i: (0, 0)),
            pl.BlockSpec((HID,), lambda i: (0,)),
            pl.BlockSpec((F_IN, F_IN), lambda i: (0, 0)),
            pl.BlockSpec((F_IN,), lambda i: (0,)),
        ],
        out_specs=[
            pl.BlockSpec((BLK, HID), lambda i: (i, 0)),
            pl.BlockSpec((8, HID), lambda i: (0, 0)),
            pl.BlockSpec((BLK, F_IN), lambda i: (i, 0)),
            pl.BlockSpec((BLK, 8), lambda i: (i, 0)),
        ],
        out_shape=[
            jax.ShapeDtypeStruct((N, HID), jnp.float32),
            jax.ShapeDtypeStruct((8, HID), jnp.float32),
            jax.ShapeDtypeStruct((N, F_IN), jnp.float32),
            jax.ShapeDtypeStruct((N, 8), jnp.float32),
        ],
    )(x, w_enc, b_enc, wa, ba)


def _tc2_body(y_ref, stats_ref, g_ref, be_ref, w1_ref, asr_ref, adr_ref,
              c_ref, h_ref, aa_ref, m_ref):
    i = pl.program_id(0)
    st = stats_ref[...]
    mu = st[0] / N
    var = st[1] / N - mu * mu
    inv = lax.rsqrt(var + 1e-5)
    c = _elu((y_ref[...] - mu) * inv * g_ref[...] + be_ref[...])
    c_ref[...] = c
    h = jnp.dot(c, w1_ref[...], preferred_element_type=jnp.float32)
    h_ref[...] = h
    asr = asr_ref[...]
    adr = adr_ref[...]
    as0 = jnp.sum(h[:, :HID] * asr[0], axis=1)
    as1 = jnp.sum(h[:, HID:] * asr[1], axis=1)
    ad0 = jnp.sum(h[:, :HID] * adr[0], axis=1)
    ad1 = jnp.sum(h[:, HID:] * adr[1], axis=1)
    aa_ref[...] = jnp.stack([as0, as1, ad0, ad1], axis=1)
    mblk = jnp.full((8, 128), jnp.maximum(jnp.max(as0), jnp.max(as1)),
                    jnp.float32)

    @pl.when(i == 0)
    def _():
        m_ref[...] = mblk

    @pl.when(i != 0)
    def _():
        m_ref[...] = jnp.maximum(m_ref[...], mblk)


def _tc2(y, stats, gamma, beta, w1, a_src, a_dst):
    return pl.pallas_call(
        _tc2_body,
        grid=(N // BLK,),
        in_specs=[
            pl.BlockSpec((BLK, HID), lambda i: (i, 0)),
            pl.BlockSpec((8, HID), lambda i: (0, 0)),
            pl.BlockSpec((HID,), lambda i: (0,)),
            pl.BlockSpec((HID,), lambda i: (0,)),
            pl.BlockSpec((HID, D), lambda i: (0, 0)),
            pl.BlockSpec((HEADS, HID), lambda i: (0, 0)),
            pl.BlockSpec((HEADS, HID), lambda i: (0, 0)),
        ],
        out_specs=[
            pl.BlockSpec((BLK, HID), lambda i: (i, 0)),
            pl.BlockSpec((BLK, D), lambda i: (i, 0)),
            pl.BlockSpec((BLK, 4), lambda i: (i, 0)),
            pl.BlockSpec((8, 128), lambda i: (0, 0)),
        ],
        out_shape=[
            jax.ShapeDtypeStruct((N, HID), jnp.float32),
            jax.ShapeDtypeStruct((N, D), jnp.float32),
            jax.ShapeDtypeStruct((N, 4), jnp.float32),
            jax.ShapeDtypeStruct((8, 128), jnp.float32),
        ],
    )(y, stats, gamma, beta, w1, a_src, a_dst)


def _gat_post(nd_ref, bias_ref):
    nd = nd_ref[...]
    h0 = nd[0, :, :HID] / nd[0, :, HID:HID + 1]
    h1 = nd[1, :, :HID] / nd[1, :, HID:HID + 1]
    return _elu(jnp.concatenate([h0, h1], axis=1) + bias_ref[...])


def _tc3_body(nd_ref, b1_ref, c_ref, w2_ref, asr_ref, adr_ref,
              cb_ref, h_ref, aa_ref, m_ref):
    i = pl.program_id(0)
    h_i = _gat_post(nd_ref, b1_ref)
    c = c_ref[...]
    cexp = jnp.concatenate([c, c], axis=1)
    cb = cexp - h_i
    cb_ref[...] = cb
    h = jnp.dot(cb, w2_ref[...], preferred_element_type=jnp.float32)
    h_ref[...] = h
    asr = asr_ref[...]
    adr = adr_ref[...]
    as0 = jnp.sum(h[:, :HID] * asr[0], axis=1)
    as1 = jnp.sum(h[:, HID:] * asr[1], axis=1)
    ad0 = jnp.sum(h[:, :HID] * adr[0], axis=1)
    ad1 = jnp.sum(h[:, HID:] * adr[1], axis=1)
    aa_ref[...] = jnp.stack([as0, as1, ad0, ad1], axis=1)
    mblk = jnp.full((8, 128), jnp.maximum(jnp.max(as0), jnp.max(as1)),
                    jnp.float32)

    @pl.when(i == 0)
    def _():
        m_ref[...] = mblk

    @pl.when(i != 0)
    def _():
        m_ref[...] = jnp.maximum(m_ref[...], mblk)


def _tc3(nd, bias1, c, w2, a_src, a_dst):
    return pl.pallas_call(
        _tc3_body,
        grid=(N // BLK,),
        in_specs=[
            pl.BlockSpec((2, BLK, HID + 16), lambda i: (0, i, 0)),
            pl.BlockSpec((D,), lambda i: (0,)),
            pl.BlockSpec((BLK, HID), lambda i: (i, 0)),
            pl.BlockSpec((D, D), lambda i: (0, 0)),
            pl.BlockSpec((HEADS, HID), lambda i: (0, 0)),
            pl.BlockSpec((HEADS, HID), lambda i: (0, 0)),
        ],
        out_specs=[
            pl.BlockSpec((BLK, D), lambda i: (i, 0)),
            pl.BlockSpec((BLK, D), lambda i: (i, 0)),
            pl.BlockSpec((BLK, 4), lambda i: (i, 0)),
            pl.BlockSpec((8, 128), lambda i: (0, 0)),
        ],
        out_shape=[
            jax.ShapeDtypeStruct((N, D), jnp.float32),
            jax.ShapeDtypeStruct((N, D), jnp.float32),
            jax.ShapeDtypeStruct((N, 4), jnp.float32),
            jax.ShapeDtypeStruct((8, 128), jnp.float32),
        ],
    )(nd, bias1, c, w2, a_src, a_dst)


def _tc4_body(nd_ref, b2_ref, c_ref, cb_ref, wf1_ref, bf1_ref,
              wf2_ref, bf2_ref, df_ref):
    h_u = _gat_post(nd_ref, b2_ref)
    cb = cb_ref[...]
    cbu = cb - h_u
    c = c_ref[...]
    cexp = jnp.concatenate([c, c], axis=1)
    wf1 = wf1_ref[...]
    z = (jnp.dot(cexp, wf1[:D], preferred_element_type=jnp.float32)
         + jnp.dot(cb, wf1[D:2 * D], preferred_element_type=jnp.float32)
         + jnp.dot(cbu, wf1[2 * D:], preferred_element_type=jnp.float32)
         + bf1_ref[...])
    z = _lrelu(z)
    df = jnp.sum(z * wf2_ref[...], axis=1) + bf2_ref[0]
    df_ref[...] = jnp.broadcast_to(df[:, None], (BLK, 8))


def _tc4(nd, bias2, c, cb, wf1, bf1, wf2r, bf2):
    return pl.pallas_call(
        _tc4_body,
        grid=(N // BLK,),
        in_specs=[
            pl.BlockSpec((2, BLK, HID + 16), lambda i: (0, i, 0)),
            pl.BlockSpec((D,), lambda i: (0,)),
            pl.BlockSpec((BLK, HID), lambda i: (i, 0)),
            pl.BlockSpec((BLK, D), lambda i: (i, 0)),
            pl.BlockSpec((3 * D, HID), lambda i: (0, 0)),
            pl.BlockSpec((HID,), lambda i: (0,)),
            pl.BlockSpec((1, HID), lambda i: (0, 0)),
            pl.BlockSpec((1,), lambda i: (0,)),
        ],
        out_specs=[pl.BlockSpec((BLK, 8), lambda i: (i, 0))],
        out_shape=[jax.ShapeDtypeStruct((N, 8), jnp.float32)],
    )(nd, bias2, c, cb, wf1, bf1, wf2r, bf2)


def _sc_gat(h_pad, aa2, m2, src3, dst3, nch):
    mesh = plsc.VectorSubcoreMesh(core_axis_name="c", subcore_axis_name="s",
                                  num_cores=NC, num_subcores=NS)

    W = HID + 16

    @functools.partial(
        pl.kernel,
        out_type=jax.ShapeDtypeStruct((NC, NP, W), jnp.float32),
        mesh=mesh,
        compiler_params=pltpu.CompilerParams(needs_layout_passes=False,
                                             use_tc_tiling_on_sc=False),
        scratch_types=[
            pltpu.VMEM((SB, CH), jnp.int32),
            pltpu.VMEM((SB, CH), jnp.int32),
            pltpu.VMEM((NP * 2,), jnp.float32),
            pltpu.VMEM((16,), jnp.float32),
            pltpu.VMEM((2, CH, HID), jnp.float32),
            pltpu.VMEM((2, CH, W), jnp.float32),
            pltpu.VMEM_SHARED((NP, W), jnp.float32),
            pltpu.SemaphoreType.DMA,
            pltpu.SemaphoreType.DMA,
            pltpu.SemaphoreType.DMA,
            pltpu.SemaphoreType.DMA,
        ],
    )
    def k(h_hbm, aa_hbm, m_hbm, src_hbm, dst_hbm, out_hbm,
          srcv, dstv, aav, mv, rowsv, stgv, acc_sh, gsem0, gsem1,
          ssem0, ssem1):
        cid = lax.axis_index("c")
        sid = lax.axis_index("s")

        def zero_rows(r, _):
            for cb in range(W // L):
                stgv[0, r, pl.ds(cb * L, L)] = jnp.zeros((L,), jnp.float32)
                stgv[1, r, pl.ds(cb * L, L)] = jnp.zeros((L,), jnp.float32)
            return 0

        lax.fori_loop(0, CH, zero_rows, 0)

        base = sid * ROWS_PT
        for kk in range(ROWS_PT // CH):
            pltpu.sync_copy(stgv.at[0], acc_sh.at[pl.ds(base + kk * CH, CH), :])

        pltpu.sync_copy(aa_hbm.at[cid], aav)
        pltpu.sync_copy(m_hbm.at[cid], mv)
        m0 = mv[...][0]
        zi = jnp.zeros((L,), jnp.int32)

        plsc.subcore_barrier()

        def start_gather(ci, b):
            pltpu.async_copy(h_hbm.at[cid].at[srcv.at[ci]], rowsv.at[b],
                             gsem0 if b == 0 else gsem1)

        def wait_gather(b):
            pltpu.make_async_copy(
                h_hbm.at[cid].at[srcv.at[0]], rowsv.at[b],
                gsem0 if b == 0 else gsem1).wait()

        def wait_scatter(b):
            pltpu.make_async_copy(
                stgv.at[b], acc_sh.at[dstv.at[0]],
                ssem0 if b == 0 else ssem1).wait()

        def process(ci, b):
            for g in range(CH // L):
                sg2 = srcv[ci, pl.ds(g * L, L)] * 2
                dg2 = dstv[ci, pl.ds(g * L, L)] * 2
                asv = plsc.load_gather(aav, [sg2])
                adv = plsc.load_gather(aav, [dg2 + 1])
                w = jnp.exp(_lrelu(asv + adv) - _lrelu(m0 + adv))
                rows16 = lax.iota(jnp.int32, L) + g * L
                plsc.store_scatter(stgv.at[b], [rows16, zi + HID], w)
                for j in range(L):
                    e = g * L + j
                    for rb in range(HID // L):
                        stgv[b, e, pl.ds(rb * L, L)] = (
                            rowsv[b, e, pl.ds(rb * L, L)] * w[j])

            pltpu.async_copy(stgv.at[b], acc_sh.at[dstv.at[ci]],
                             ssem0 if b == 0 else ssem1, add=True)

        def superchunk(sci, _):
            pltpu.sync_copy(src_hbm.at[sid, pl.ds(sci * SB, SB)], srcv)
            pltpu.sync_copy(dst_hbm.at[sid, pl.ds(sci * SB, SB)], dstv)
            start_gather(0, 0)

            def pair(cj, _):
                ci0 = 2 * cj
                start_gather(ci0 + 1, 1)
                wait_gather(0)

                @pl.when((sci > 0) | (cj > 0))
                def _():
                    wait_scatter(0)

                process(ci0, 0)

                @pl.when(ci0 + 2 < SB)
                def _():
                    start_gather(ci0 + 2, 0)

                wait_gather(1)

                @pl.when((sci > 0) | (cj > 0))
                def _():
                    wait_scatter(1)

                process(ci0 + 1, 1)
                return 0

            lax.fori_loop(0, SB // 2, pair, 0)
            return 0

        lax.fori_loop(0, nch // SB, superchunk, 0)
        wait_scatter(0)
        wait_scatter(1)

        plsc.subcore_barrier()

        for kk in range(ROWS_PT // CH):
            r0 = base + kk * CH
            pltpu.sync_copy(acc_sh.at[pl.ds(r0, CH), :], stgv.at[0])
            pltpu.sync_copy(stgv.at[0], out_hbm.at[cid, pl.ds(r0, CH), :])

    return k(h_pad, aa2, m2, src3, dst3)


def _prep_edges(ei):
    e = ei.shape[1] + N
    nch = -(-e // (NS * CH))
    nch = -(-nch // SB) * SB
    epad = NS * CH * nch
    loop = jnp.arange(N, dtype=jnp.int32)
    src = jnp.concatenate(
        [ei[0].astype(jnp.int32), loop,
         jnp.full((epad - e,), NP - 1, jnp.int32)])
    dst = jnp.concatenate(
        [ei[1].astype(jnp.int32), loop,
         jnp.full((epad - e,), NP - 1, jnp.int32)])
    return src.reshape(NS, nch, CH), dst.reshape(NS, nch, CH), nch


def kernel(x_raw, edge_industry, edge_universe, W_enc, b_enc, gamma, beta,
           W1, att_src1, att_dst1, bias1, W2, att_src2, att_dst2, bias2,
           Wf1, bf1, Wf2, bf2, Wa, ba):
    y, stats, attn_weights, fe8 = _tc1(x_raw, W_enc, b_enc, Wa, ba)
    c, h1, aa1, m1 = _tc2(y, stats, gamma, beta, W1, att_src1, att_dst1)

    src1, dst1, nch1 = _prep_edges(edge_industry)
    src2, dst2, nch2 = _prep_edges(edge_universe)

    def _sc_inputs(h, aa, m):
        hp = jnp.pad(h, ((0, NP - N), (0, 0)))
        hs = jnp.stack([hp[:, :HID], hp[:, HID:]])
        aap = jnp.pad(aa, ((0, NP - N), (0, 0)))
        aa2c = jnp.stack(
            [jnp.stack([aap[:, hd], aap[:, 2 + hd]], axis=-1).reshape(-1)
             for hd in range(HEADS)])
        m2c = jnp.broadcast_to(m[0, :16][None], (2, 16))
        return hs, aa2c, m2c

    h1p, aat1, m1c = _sc_inputs(h1, aa1, m1)
    nd1 = _sc_gat(h1p, aat1, m1c, src1, dst1, nch1)

    cb, h2, aa2, m2 = _tc3(nd1[:, :N], bias1, c, W2, att_src2, att_dst2)

    h2p, aat2, m2c = _sc_inputs(h2, aa2, m2)
    nd2 = _sc_gat(h2p, aat2, m2c, src2, dst2, nch2)

    df8 = _tc4(nd2[:, :N], bias2, c, cb, Wf1, bf1,
               Wf2.reshape(1, HID), bf2)[0]

    deep_factor = df8[:, 0]
    factor_estimate = fe8[:, 0]
    return (deep_factor, factor_estimate, attn_weights)


# speedup vs baseline: 49.5599x; 1.0069x over previous
import functools

import jax
import jax.numpy as jnp
from jax import lax
from jax.experimental import pallas as pl
from jax.experimental.pallas import tpu as pltpu
from jax.experimental.pallas import tpu_sc as plsc

N = 10000
F_IN = 256
HID = 64
HEADS = 2
D = HEADS * HID

NC = 2
NS = 16
L = 16

NP = 10240
ROWS_PT = NP // NS
CH = 128
SB = 42
BLK = 1000


def _elu(x):
    return jnp.where(x > 0, x, jnp.exp(x) - 1.0)


def _lrelu(x):
    return jnp.maximum(x, 0.2 * x)


def _tc1_body(x_ref, w_ref, b_ref, wa_ref, ba_ref,
              y_ref, stats_ref, aw_ref, fe_ref):
    i = pl.program_id(0)
    x = x_ref[...]
    y = jnp.dot(x, w_ref[...], preferred_element_type=jnp.float32) + b_ref[...]
    y_ref[...] = y
    s = jnp.sum(y, axis=0)
    sq = jnp.sum(y * y, axis=0)
    st = jnp.concatenate([s[None], sq[None], jnp.zeros((6, HID), jnp.float32)], 0)

    @pl.when(i == 0)
    def _():
        stats_ref[...] = st

    @pl.when(i != 0)
    def _():
        stats_ref[...] += st

    logits = jnp.dot(x, wa_ref[...], preferred_element_type=jnp.float32) + ba_ref[...]
    m = jnp.max(logits, axis=1, keepdims=True)
    e = jnp.exp(logits - m)
    w = e / jnp.sum(e, axis=1, keepdims=True)
    aw_ref[...] = w
    fe = jnp.sum(x * w, axis=1)
    fe_ref[...] = jnp.broadcast_to(fe[:, None], (BLK, 8))


def _tc1(x, w_enc, b_enc, wa, ba):
    return pl.pallas_call(
        _tc1_body,
        grid=(N // BLK,),
        in_specs=[
            pl.BlockSpec((BLK, F_IN), lambda i: (i, 0)),
            pl.BlockSpec((F_IN, HID), lambda i: (0, 0)),
            pl.BlockSpec((HID,), lambda i: (0,)),
            pl.BlockSpec((F_IN, F_IN), lambda i: (0, 0)),
            pl.BlockSpec((F_IN,), lambda i: (0,)),
        ],
        out_specs=[
            pl.BlockSpec((BLK, HID), lambda i: (i, 0)),
            pl.BlockSpec((8, HID), lambda i: (0, 0)),
            pl.BlockSpec((BLK, F_IN), lambda i: (i, 0)),
            pl.BlockSpec((BLK, 8), lambda i: (i, 0)),
        ],
        out_shape=[
            jax.ShapeDtypeStruct((N, HID), jnp.float32),
            jax.ShapeDtypeStruct((8, HID), jnp.float32),
            jax.ShapeDtypeStruct((N, F_IN), jnp.float32),
            jax.ShapeDtypeStruct((N, 8), jnp.float32),
        ],
    )(x, w_enc, b_enc, wa, ba)


def _tc2_body(y_ref, stats_ref, g_ref, be_ref, w1_ref, asr_ref, adr_ref,
              c_ref, h_ref, aa_ref, m_ref):
    i = pl.program_id(0)
    st = stats_ref[...]
    mu = st[0] / N
    var = st[1] / N - mu * mu
    inv = lax.rsqrt(var + 1e-5)
    c = _elu((y_ref[...] - mu) * inv * g_ref[...] + be_ref[...])
    c_ref[...] = c
    h = jnp.dot(c, w1_ref[...], preferred_element_type=jnp.float32)
    h_ref[...] = h
    asr = asr_ref[...]
    adr = adr_ref[...]
    as0 = jnp.sum(h[:, :HID] * asr[0], axis=1)
    as1 = jnp.sum(h[:, HID:] * asr[1], axis=1)
    ad0 = jnp.sum(h[:, :HID] * adr[0], axis=1)
    ad1 = jnp.sum(h[:, HID:] * adr[1], axis=1)
    aa_ref[...] = jnp.stack([as0, as1, ad0, ad1], axis=1)
    mblk = jnp.full((8, 128), jnp.maximum(jnp.max(as0), jnp.max(as1)),
                    jnp.float32)

    @pl.when(i == 0)
    def _():
        m_ref[...] = mblk

    @pl.when(i != 0)
    def _():
        m_ref[...] = jnp.maximum(m_ref[...], mblk)


def _tc2(y, stats, gamma, beta, w1, a_src, a_dst):
    return pl.pallas_call(
        _tc2_body,
        grid=(N // BLK,),
        in_specs=[
            pl.BlockSpec((BLK, HID), lambda i: (i, 0)),
            pl.BlockSpec((8, HID), lambda i: (0, 0)),
            pl.BlockSpec((HID,), lambda i: (0,)),
            pl.BlockSpec((HID,), lambda i: (0,)),
            pl.BlockSpec((HID, D), lambda i: (0, 0)),
            pl.BlockSpec((HEADS, HID), lambda i: (0, 0)),
            pl.BlockSpec((HEADS, HID), lambda i: (0, 0)),
        ],
        out_specs=[
            pl.BlockSpec((BLK, HID), lambda i: (i, 0)),
            pl.BlockSpec((BLK, D), lambda i: (i, 0)),
            pl.BlockSpec((BLK, 4), lambda i: (i, 0)),
            pl.BlockSpec((8, 128), lambda i: (0, 0)),
        ],
        out_shape=[
            jax.ShapeDtypeStruct((N, HID), jnp.float32),
            jax.ShapeDtypeStruct((N, D), jnp.float32),
            jax.ShapeDtypeStruct((N, 4), jnp.float32),
            jax.ShapeDtypeStruct((8, 128), jnp.float32),
        ],
    )(y, stats, gamma, beta, w1, a_src, a_dst)


def _gat_post(nd_ref, bias_ref):
    nd = nd_ref[...]
    h0 = nd[0, :, :HID] / nd[0, :, HID:HID + 1]
    h1 = nd[1, :, :HID] / nd[1, :, HID:HID + 1]
    return _elu(jnp.concatenate([h0, h1], axis=1) + bias_ref[...])


def _tc3_body(nd_ref, b1_ref, c_ref, w2_ref, asr_ref, adr_ref,
              cb_ref, h_ref, aa_ref, m_ref):
    i = pl.program_id(0)
    h_i = _gat_post(nd_ref, b1_ref)
    c = c_ref[...]
    cexp = jnp.concatenate([c, c], axis=1)
    cb = cexp - h_i
    cb_ref[...] = cb
    h = jnp.dot(cb, w2_ref[...], preferred_element_type=jnp.float32)
    h_ref[...] = h
    asr = asr_ref[...]
    adr = adr_ref[...]
    as0 = jnp.sum(h[:, :HID] * asr[0], axis=1)
    as1 = jnp.sum(h[:, HID:] * asr[1], axis=1)
    ad0 = jnp.sum(h[:, :HID] * adr[0], axis=1)
    ad1 = jnp.sum(h[:, HID:] * adr[1], axis=1)
    aa_ref[...] = jnp.stack([as0, as1, ad0, ad1], axis=1)
    mblk = jnp.full((8, 128), jnp.maximum(jnp.max(as0), jnp.max(as1)),
                    jnp.float32)

    @pl.when(i == 0)
    def _():
        m_ref[...] = mblk

    @pl.when(i != 0)
    def _():
        m_ref[...] = jnp.maximum(m_ref[...], mblk)


def _tc3(nd, bias1, c, w2, a_src, a_dst):
    return pl.pallas_call(
        _tc3_body,
        grid=(N // BLK,),
        in_specs=[
            pl.BlockSpec((2, BLK, HID + 16), lambda i: (0, i, 0)),
            pl.BlockSpec((D,), lambda i: (0,)),
            pl.BlockSpec((BLK, HID), lambda i: (i, 0)),
            pl.BlockSpec((D, D), lambda i: (0, 0)),
            pl.BlockSpec((HEADS, HID), lambda i: (0, 0)),
            pl.BlockSpec((HEADS, HID), lambda i: (0, 0)),
        ],
        out_specs=[
            pl.BlockSpec((BLK, D), lambda i: (i, 0)),
            pl.BlockSpec((BLK, D), lambda i: (i, 0)),
            pl.BlockSpec((BLK, 4), lambda i: (i, 0)),
            pl.BlockSpec((8, 128), lambda i: (0, 0)),
        ],
        out_shape=[
            jax.ShapeDtypeStruct((N, D), jnp.float32),
            jax.ShapeDtypeStruct((N, D), jnp.float32),
            jax.ShapeDtypeStruct((N, 4), jnp.float32),
            jax.ShapeDtypeStruct((8, 128), jnp.float32),
        ],
    )(nd, bias1, c, w2, a_src, a_dst)


def _tc4_body(nd_ref, b2_ref, c_ref, cb_ref, wf1_ref, bf1_ref,
              wf2_ref, bf2_ref, df_ref):
    h_u = _gat_post(nd_ref, b2_ref)
    cb = cb_ref[...]
    cbu = cb - h_u
    c = c_ref[...]
    cexp = jnp.concatenate([c, c], axis=1)
    wf1 = wf1_ref[...]
    z = (jnp.dot(cexp, wf1[:D], preferred_element_type=jnp.float32)
         + jnp.dot(cb, wf1[D:2 * D], preferred_element_type=jnp.float32)
         + jnp.dot(cbu, wf1[2 * D:], preferred_element_type=jnp.float32)
         + bf1_ref[...])
    z = _lrelu(z)
    df = jnp.sum(z * wf2_ref[...], axis=1) + bf2_ref[0]
    df_ref[...] = jnp.broadcast_to(df[:, None], (BLK, 8))


def _tc4(nd, bias2, c, cb, wf1, bf1, wf2r, bf2):
    return pl.pallas_call(
        _tc4_body,
        grid=(N // BLK,),
        in_specs=[
            pl.BlockSpec((2, BLK, HID + 16), lambda i: (0, i, 0)),
            pl.BlockSpec((D,), lambda i: (0,)),
            pl.BlockSpec((BLK, HID), lambda i: (i, 0)),
            pl.BlockSpec((BLK, D), lambda i: (i, 0)),
            pl.BlockSpec((3 * D, HID), lambda i: (0, 0)),
            pl.BlockSpec((HID,), lambda i: (0,)),
            pl.BlockSpec((1, HID), lambda i: (0, 0)),
            pl.BlockSpec((1,), lambda i: (0,)),
        ],
        out_specs=[pl.BlockSpec((BLK, 8), lambda i: (i, 0))],
        out_shape=[jax.ShapeDtypeStruct((N, 8), jnp.float32)],
    )(nd, bias2, c, cb, wf1, bf1, wf2r, bf2)


def _sc_gat(h_pad, aa2, m2, src3, dst3, nch):
    mesh = plsc.VectorSubcoreMesh(core_axis_name="c", subcore_axis_name="s",
                                  num_cores=NC, num_subcores=NS)

    W = HID + 16

    @functools.partial(
        pl.kernel,
        out_type=jax.ShapeDtypeStruct((NC, NP, W), jnp.float32),
        mesh=mesh,
        compiler_params=pltpu.CompilerParams(needs_layout_passes=False,
                                             use_tc_tiling_on_sc=False),
        scratch_types=[
            pltpu.VMEM((SB, CH), jnp.int32),
            pltpu.VMEM((SB, CH), jnp.int32),
            pltpu.VMEM((NP * 2,), jnp.float32),
            pltpu.VMEM((16,), jnp.float32),
            pltpu.VMEM((2, CH, HID), jnp.float32),
            pltpu.VMEM((2, CH, W), jnp.float32),
            pltpu.VMEM_SHARED((NP, W), jnp.float32),
            pltpu.SemaphoreType.DMA,
            pltpu.SemaphoreType.DMA,
            pltpu.SemaphoreType.DMA,
            pltpu.SemaphoreType.DMA,
        ],
    )
    def k(h_hbm, aa_hbm, m_hbm, src_hbm, dst_hbm, out_hbm,
          srcv, dstv, aav, mv, rowsv, stgv, acc_sh, gsem0, gsem1,
          ssem0, ssem1):
        cid = lax.axis_index("c")
        sid = lax.axis_index("s")

        def zero_rows(r, _):
            for cb in range(W // L):
                stgv[0, r, pl.ds(cb * L, L)] = jnp.zeros((L,), jnp.float32)
                stgv[1, r, pl.ds(cb * L, L)] = jnp.zeros((L,), jnp.float32)
            return 0

        lax.fori_loop(0, CH, zero_rows, 0)

        base = sid * ROWS_PT
        for kk in range(ROWS_PT // CH):
            pltpu.sync_copy(stgv.at[0], acc_sh.at[pl.ds(base + kk * CH, CH), :])

        pltpu.sync_copy(aa_hbm.at[cid], aav)
        pltpu.sync_copy(m_hbm.at[cid], mv)
        m0 = mv[...][0]
        zi = jnp.zeros((L,), jnp.int32)

        plsc.subcore_barrier()

        def start_gather(ci, b):
            pltpu.async_copy(h_hbm.at[cid].at[srcv.at[ci]], rowsv.at[b],
                             gsem0 if b == 0 else gsem1)

        def wait_gather(b):
            pltpu.make_async_copy(
                h_hbm.at[cid].at[srcv.at[0]], rowsv.at[b],
                gsem0 if b == 0 else gsem1).wait()

        def wait_scatter(b):
            pltpu.make_async_copy(
                stgv.at[b], acc_sh.at[dstv.at[0]],
                ssem0 if b == 0 else ssem1).wait()

        def process(ci, b):
            for g in range(CH // L):
                sg2 = srcv[ci, pl.ds(g * L, L)] * 2
                dg2 = dstv[ci, pl.ds(g * L, L)] * 2
                asv = plsc.load_gather(aav, [sg2])
                adv = plsc.load_gather(aav, [dg2 + 1])
                w = jnp.exp(_lrelu(asv + adv) - _lrelu(m0 + adv))
                rows16 = lax.iota(jnp.int32, L) + g * L
                plsc.store_scatter(stgv.at[b], [rows16, zi + HID], w)
                for j in range(L):
                    e = g * L + j
                    for rb in range(HID // L):
                        stgv[b, e, pl.ds(rb * L, L)] = (
                            rowsv[b, e, pl.ds(rb * L, L)] * w[j])

            pltpu.async_copy(stgv.at[b], acc_sh.at[dstv.at[ci]],
                             ssem0 if b == 0 else ssem1, add=True)

        def superchunk(sci, _):
            pltpu.sync_copy(src_hbm.at[sid, pl.ds(sci * SB, SB)], srcv)
            pltpu.sync_copy(dst_hbm.at[sid, pl.ds(sci * SB, SB)], dstv)
            start_gather(0, 0)

            def pair(cj, _):
                ci0 = 2 * cj
                start_gather(ci0 + 1, 1)
                wait_gather(0)

                @pl.when((sci > 0) | (cj > 0))
                def _():
                    wait_scatter(0)

                process(ci0, 0)

                @pl.when(ci0 + 2 < SB)
                def _():
                    start_gather(ci0 + 2, 0)

                wait_gather(1)

                @pl.when((sci > 0) | (cj > 0))
                def _():
                    wait_scatter(1)

                process(ci0 + 1, 1)
                return 0

            lax.fori_loop(0, SB // 2, pair, 0)
            return 0

        lax.fori_loop(0, nch // SB, superchunk, 0)
        wait_scatter(0)
        wait_scatter(1)

        plsc.subcore_barrier()

        for kk in range(ROWS_PT // CH):
            r0 = base + kk * CH
            pltpu.sync_copy(acc_sh.at[pl.ds(r0, CH), :], stgv.at[0])
            pltpu.sync_copy(stgv.at[0], out_hbm.at[cid, pl.ds(r0, CH), :])

    return k(h_pad, aa2, m2, src3, dst3)


def _prep_edges(ei):
    e = ei.shape[1] + N
    nch = -(-e // (NS * CH))
    nch = -(-nch // SB) * SB
    epad = NS * CH * nch
    loop = jnp.arange(N, dtype=jnp.int32)
    src = jnp.concatenate(
        [ei[0].astype(jnp.int32), loop,
         jnp.full((epad - e,), NP - 1, jnp.int32)])
    dst = jnp.concatenate(
        [ei[1].astype(jnp.int32), loop,
         jnp.full((epad - e,), NP - 1, jnp.int32)])
    return src.reshape(NS, nch, CH), dst.reshape(NS, nch, CH), nch


def kernel(x_raw, edge_industry, edge_universe, W_enc, b_enc, gamma, beta,
           W1, att_src1, att_dst1, bias1, W2, att_src2, att_dst2, bias2,
           Wf1, bf1, Wf2, bf2, Wa, ba):
    y, stats, attn_weights, fe8 = _tc1(x_raw, W_enc, b_enc, Wa, ba)
    c, h1, aa1, m1 = _tc2(y, stats, gamma, beta, W1, att_src1, att_dst1)

    src1, dst1, nch1 = _prep_edges(edge_industry)
    src2, dst2, nch2 = _prep_edges(edge_universe)

    def _sc_inputs(h, aa, m):
        hp = jnp.pad(h, ((0, NP - N), (0, 0)))
        hs = jnp.stack([hp[:, :HID], hp[:, HID:]])
        aap = jnp.pad(aa, ((0, NP - N), (0, 0)))
        aa2c = jnp.stack(
            [jnp.stack([aap[:, hd], aap[:, 2 + hd]], axis=-1).reshape(-1)
             for hd in range(HEADS)])
        m2c = jnp.broadcast_to(m[0, :16][None], (2, 16))
        return hs, aa2c, m2c

    h1p, aat1, m1c = _sc_inputs(h1, aa1, m1)
    nd1 = _sc_gat(h1p, aat1, m1c, src1, dst1, nch1)

    cb, h2, aa2, m2 = _tc3(nd1[:, :N], bias1, c, W2, att_src2, att_dst2)

    h2p, aat2, m2c = _sc_inputs(h2, aa2, m2)
    nd2 = _sc_gat(h2p, aat2, m2c, src2, dst2, nch2)

    df8 = _tc4(nd2[:, :N], bias2, c, cb, Wf1, bf1,
               Wf2.reshape(1, HID), bf2)[0]

    deep_factor = df8[:, 0]
    factor_estimate = fe8[:, 0]
    return (deep_factor, factor_estimate, attn_weights)


# speedup vs baseline: 49.9545x; 1.0080x over previous
import functools

import jax
import jax.numpy as jnp
from jax import lax
from jax.experimental import pallas as pl
from jax.experimental.pallas import tpu as pltpu
from jax.experimental.pallas import tpu_sc as plsc

N = 10000
F_IN = 256
HID = 64
HEADS = 2
D = HEADS * HID

NC = 2
NS = 16
L = 16

NP = 10240
ROWS_PT = NP // NS
CH = 128
SB = 84
BLK = 1000


def _elu(x):
    return jnp.where(x > 0, x, jnp.exp(x) - 1.0)


def _lrelu(x):
    return jnp.maximum(x, 0.2 * x)


def _tc1_body(x_ref, w_ref, b_ref, wa_ref, ba_ref,
              y_ref, stats_ref, aw_ref, fe_ref):
    i = pl.program_id(0)
    x = x_ref[...]
    y = jnp.dot(x, w_ref[...], preferred_element_type=jnp.float32) + b_ref[...]
    y_ref[...] = y
    s = jnp.sum(y, axis=0)
    sq = jnp.sum(y * y, axis=0)
    st = jnp.concatenate([s[None], sq[None], jnp.zeros((6, HID), jnp.float32)], 0)

    @pl.when(i == 0)
    def _():
        stats_ref[...] = st

    @pl.when(i != 0)
    def _():
        stats_ref[...] += st

    logits = jnp.dot(x, wa_ref[...], preferred_element_type=jnp.float32) + ba_ref[...]
    m = jnp.max(logits, axis=1, keepdims=True)
    e = jnp.exp(logits - m)
    w = e / jnp.sum(e, axis=1, keepdims=True)
    aw_ref[...] = w
    fe = jnp.sum(x * w, axis=1)
    fe_ref[...] = jnp.broadcast_to(fe[:, None], (BLK, 8))


def _tc1(x, w_enc, b_enc, wa, ba):
    return pl.pallas_call(
        _tc1_body,
        grid=(N // BLK,),
        in_specs=[
            pl.BlockSpec((BLK, F_IN), lambda i: (i, 0)),
            pl.BlockSpec((F_IN, HID), lambda i: (0, 0)),
            pl.BlockSpec((HID,), lambda i: (0,)),
            pl.BlockSpec((F_IN, F_IN), lambda i: (0, 0)),
            pl.BlockSpec((F_IN,), lambda i: (0,)),
        ],
        out_specs=[
            pl.BlockSpec((BLK, HID), lambda i: (i, 0)),
            pl.BlockSpec((8, HID), lambda i: (0, 0)),
            pl.BlockSpec((BLK, F_IN), lambda i: (i, 0)),
            pl.BlockSpec((BLK, 8), lambda i: (i, 0)),
        ],
        out_shape=[
            jax.ShapeDtypeStruct((N, HID), jnp.float32),
            jax.ShapeDtypeStruct((8, HID), jnp.float32),
            jax.ShapeDtypeStruct((N, F_IN), jnp.float32),
            jax.ShapeDtypeStruct((N, 8), jnp.float32),
        ],
    )(x, w_enc, b_enc, wa, ba)


def _tc2_body(y_ref, stats_ref, g_ref, be_ref, w1_ref, asr_ref, adr_ref,
              c_ref, h_ref, aa_ref, m_ref):
    i = pl.program_id(0)
    st = stats_ref[...]
    mu = st[0] / N
    var = st[1] / N - mu * mu
    inv = lax.rsqrt(var + 1e-5)
    c = _elu((y_ref[...] - mu) * inv * g_ref[...] + be_ref[...])
    c_ref[...] = c
    h = jnp.dot(c, w1_ref[...], preferred_element_type=jnp.float32)
    h_ref[...] = h
    asr = asr_ref[...]
    adr = adr_ref[...]
    as0 = jnp.sum(h[:, :HID] * asr[0], axis=1)
    as1 = jnp.sum(h[:, HID:] * asr[1], axis=1)
    ad0 = jnp.sum(h[:, :HID] * adr[0], axis=1)
    ad1 = jnp.sum(h[:, HID:] * adr[1], axis=1)
    aa_ref[...] = jnp.stack([as0, as1, ad0, ad1], axis=1)
    mblk = jnp.full((8, 128), jnp.maximum(jnp.max(as0), jnp.max(as1)),
                    jnp.float32)

    @pl.when(i == 0)
    def _():
        m_ref[...] = mblk

    @pl.when(i != 0)
    def _():
        m_ref[...] = jnp.maximum(m_ref[...], mblk)


def _tc2(y, stats, gamma, beta, w1, a_src, a_dst):
    return pl.pallas_call(
        _tc2_body,
        grid=(N // BLK,),
        in_specs=[
            pl.BlockSpec((BLK, HID), lambda i: (i, 0)),
            pl.BlockSpec((8, HID), lambda i: (0, 0)),
            pl.BlockSpec((HID,), lambda i: (0,)),
            pl.BlockSpec((HID,), lambda i: (0,)),
            pl.BlockSpec((HID, D), lambda i: (0, 0)),
            pl.BlockSpec((HEADS, HID), lambda i: (0, 0)),
            pl.BlockSpec((HEADS, HID), lambda i: (0, 0)),
        ],
        out_specs=[
            pl.BlockSpec((BLK, HID), lambda i: (i, 0)),
            pl.BlockSpec((BLK, D), lambda i: (i, 0)),
            pl.BlockSpec((BLK, 4), lambda i: (i, 0)),
            pl.BlockSpec((8, 128), lambda i: (0, 0)),
        ],
        out_shape=[
            jax.ShapeDtypeStruct((N, HID), jnp.float32),
            jax.ShapeDtypeStruct((N, D), jnp.float32),
            jax.ShapeDtypeStruct((N, 4), jnp.float32),
            jax.ShapeDtypeStruct((8, 128), jnp.float32),
        ],
    )(y, stats, gamma, beta, w1, a_src, a_dst)


def _gat_post(nd_ref, bias_ref):
    nd = nd_ref[...]
    h0 = nd[0, :, :HID] / nd[0, :, HID:HID + 1]
    h1 = nd[1, :, :HID] / nd[1, :, HID:HID + 1]
    return _elu(jnp.concatenate([h0, h1], axis=1) + bias_ref[...])


def _tc3_body(nd_ref, b1_ref, c_ref, w2_ref, asr_ref, adr_ref,
              cb_ref, h_ref, aa_ref, m_ref):
    i = pl.program_id(0)
    h_i = _gat_post(nd_ref, b1_ref)
    c = c_ref[...]
    cexp = jnp.concatenate([c, c], axis=1)
    cb = cexp - h_i
    cb_ref[...] = cb
    h = jnp.dot(cb, w2_ref[...], preferred_element_type=jnp.float32)
    h_ref[...] = h
    asr = asr_ref[...]
    adr = adr_ref[...]
    as0 = jnp.sum(h[:, :HID] * asr[0], axis=1)
    as1 = jnp.sum(h[:, HID:] * asr[1], axis=1)
    ad0 = jnp.sum(h[:, :HID] * adr[0], axis=1)
    ad1 = jnp.sum(h[:, HID:] * adr[1], axis=1)
    aa_ref[...] = jnp.stack([as0, as1, ad0, ad1], axis=1)
    mblk = jnp.full((8, 128), jnp.maximum(jnp.max(as0), jnp.max(as1)),
                    jnp.float32)

    @pl.when(i == 0)
    def _():
        m_ref[...] = mblk

    @pl.when(i != 0)
    def _():
        m_ref[...] = jnp.maximum(m_ref[...], mblk)


def _tc3(nd, bias1, c, w2, a_src, a_dst):
    return pl.pallas_call(
        _tc3_body,
        grid=(N // BLK,),
        in_specs=[
            pl.BlockSpec((2, BLK, HID + 16), lambda i: (0, i, 0)),
            pl.BlockSpec((D,), lambda i: (0,)),
            pl.BlockSpec((BLK, HID), lambda i: (i, 0)),
            pl.BlockSpec((D, D), lambda i: (0, 0)),
            pl.BlockSpec((HEADS, HID), lambda i: (0, 0)),
            pl.BlockSpec((HEADS, HID), lambda i: (0, 0)),
        ],
        out_specs=[
            pl.BlockSpec((BLK, D), lambda i: (i, 0)),
            pl.BlockSpec((BLK, D), lambda i: (i, 0)),
            pl.BlockSpec((BLK, 4), lambda i: (i, 0)),
            pl.BlockSpec((8, 128), lambda i: (0, 0)),
        ],
        out_shape=[
            jax.ShapeDtypeStruct((N, D), jnp.float32),
            jax.ShapeDtypeStruct((N, D), jnp.float32),
            jax.ShapeDtypeStruct((N, 4), jnp.float32),
            jax.ShapeDtypeStruct((8, 128), jnp.float32),
        ],
    )(nd, bias1, c, w2, a_src, a_dst)


def _tc4_body(nd_ref, b2_ref, c_ref, cb_ref, wf1_ref, bf1_ref,
              wf2_ref, bf2_ref, df_ref):
    h_u = _gat_post(nd_ref, b2_ref)
    cb = cb_ref[...]
    cbu = cb - h_u
    c = c_ref[...]
    cexp = jnp.concatenate([c, c], axis=1)
    wf1 = wf1_ref[...]
    z = (jnp.dot(cexp, wf1[:D], preferred_element_type=jnp.float32)
         + jnp.dot(cb, wf1[D:2 * D], preferred_element_type=jnp.float32)
         + jnp.dot(cbu, wf1[2 * D:], preferred_element_type=jnp.float32)
         + bf1_ref[...])
    z = _lrelu(z)
    df = jnp.sum(z * wf2_ref[...], axis=1) + bf2_ref[0]
    df_ref[...] = jnp.broadcast_to(df[:, None], (BLK, 8))


def _tc4(nd, bias2, c, cb, wf1, bf1, wf2r, bf2):
    return pl.pallas_call(
        _tc4_body,
        grid=(N // BLK,),
        in_specs=[
            pl.BlockSpec((2, BLK, HID + 16), lambda i: (0, i, 0)),
            pl.BlockSpec((D,), lambda i: (0,)),
            pl.BlockSpec((BLK, HID), lambda i: (i, 0)),
            pl.BlockSpec((BLK, D), lambda i: (i, 0)),
            pl.BlockSpec((3 * D, HID), lambda i: (0, 0)),
            pl.BlockSpec((HID,), lambda i: (0,)),
            pl.BlockSpec((1, HID), lambda i: (0, 0)),
            pl.BlockSpec((1,), lambda i: (0,)),
        ],
        out_specs=[pl.BlockSpec((BLK, 8), lambda i: (i, 0))],
        out_shape=[jax.ShapeDtypeStruct((N, 8), jnp.float32)],
    )(nd, bias2, c, cb, wf1, bf1, wf2r, bf2)


def _sc_gat(h_pad, aa2, m2, src3, dst3, nch):
    mesh = plsc.VectorSubcoreMesh(core_axis_name="c", subcore_axis_name="s",
                                  num_cores=NC, num_subcores=NS)

    W = HID + 16

    @functools.partial(
        pl.kernel,
        out_type=jax.ShapeDtypeStruct((NC, NP, W), jnp.float32),
        mesh=mesh,
        compiler_params=pltpu.CompilerParams(needs_layout_passes=False,
                                             use_tc_tiling_on_sc=False),
        scratch_types=[
            pltpu.VMEM((SB, CH), jnp.int32),
            pltpu.VMEM((SB, CH), jnp.int32),
            pltpu.VMEM((NP * 2,), jnp.float32),
            pltpu.VMEM((16,), jnp.float32),
            pltpu.VMEM((2, CH, HID), jnp.float32),
            pltpu.VMEM((2, CH, W), jnp.float32),
            pltpu.VMEM_SHARED((NP, W), jnp.float32),
            pltpu.SemaphoreType.DMA,
            pltpu.SemaphoreType.DMA,
            pltpu.SemaphoreType.DMA,
            pltpu.SemaphoreType.DMA,
        ],
    )
    def k(h_hbm, aa_hbm, m_hbm, src_hbm, dst_hbm, out_hbm,
          srcv, dstv, aav, mv, rowsv, stgv, acc_sh, gsem0, gsem1,
          ssem0, ssem1):
        cid = lax.axis_index("c")
        sid = lax.axis_index("s")

        def zero_rows(r, _):
            for cb in range(W // L):
                stgv[0, r, pl.ds(cb * L, L)] = jnp.zeros((L,), jnp.float32)
                stgv[1, r, pl.ds(cb * L, L)] = jnp.zeros((L,), jnp.float32)
            return 0

        lax.fori_loop(0, CH, zero_rows, 0)

        base = sid * ROWS_PT
        for kk in range(ROWS_PT // CH):
            pltpu.sync_copy(stgv.at[0], acc_sh.at[pl.ds(base + kk * CH, CH), :])

        pltpu.sync_copy(aa_hbm.at[cid], aav)
        pltpu.sync_copy(m_hbm.at[cid], mv)
        m0 = mv[...][0]
        zi = jnp.zeros((L,), jnp.int32)

        plsc.subcore_barrier()

        def start_gather(ci, b):
            pltpu.async_copy(h_hbm.at[cid].at[srcv.at[ci]], rowsv.at[b],
                             gsem0 if b == 0 else gsem1)

        def wait_gather(b):
            pltpu.make_async_copy(
                h_hbm.at[cid].at[srcv.at[0]], rowsv.at[b],
                gsem0 if b == 0 else gsem1).wait()

        def wait_scatter(b):
            pltpu.make_async_copy(
                stgv.at[b], acc_sh.at[dstv.at[0]],
                ssem0 if b == 0 else ssem1).wait()

        def process(ci, b):
            for g in range(CH // L):
                sg2 = srcv[ci, pl.ds(g * L, L)] * 2
                dg2 = dstv[ci, pl.ds(g * L, L)] * 2
                asv = plsc.load_gather(aav, [sg2])
                adv = plsc.load_gather(aav, [dg2 + 1])
                w = jnp.exp(_lrelu(asv + adv) - _lrelu(m0 + adv))
                rows16 = lax.iota(jnp.int32, L) + g * L
                plsc.store_scatter(stgv.at[b], [rows16, zi + HID], w)
                for j in range(L):
                    e = g * L + j
                    for rb in range(HID // L):
                        stgv[b, e, pl.ds(rb * L, L)] = (
                            rowsv[b, e, pl.ds(rb * L, L)] * w[j])

            pltpu.async_copy(stgv.at[b], acc_sh.at[dstv.at[ci]],
                             ssem0 if b == 0 else ssem1, add=True)

        def superchunk(sci, _):
            pltpu.sync_copy(src_hbm.at[sid, pl.ds(sci * SB, SB)], srcv)
            pltpu.sync_copy(dst_hbm.at[sid, pl.ds(sci * SB, SB)], dstv)
            start_gather(0, 0)

            def pair(cj, _):
                ci0 = 2 * cj
                start_gather(ci0 + 1, 1)
                wait_gather(0)

                @pl.when((sci > 0) | (cj > 0))
                def _():
                    wait_scatter(0)

                process(ci0, 0)

                @pl.when(ci0 + 2 < SB)
                def _():
                    start_gather(ci0 + 2, 0)

                wait_gather(1)

                @pl.when((sci > 0) | (cj > 0))
                def _():
                    wait_scatter(1)

                process(ci0 + 1, 1)
                return 0

            lax.fori_loop(0, SB // 2, pair, 0)
            return 0

        lax.fori_loop(0, nch // SB, superchunk, 0)
        wait_scatter(0)
        wait_scatter(1)

        plsc.subcore_barrier()

        for kk in range(ROWS_PT // CH):
            r0 = base + kk * CH
            pltpu.sync_copy(acc_sh.at[pl.ds(r0, CH), :], stgv.at[0])
            pltpu.sync_copy(stgv.at[0], out_hbm.at[cid, pl.ds(r0, CH), :])

    return k(h_pad, aa2, m2, src3, dst3)


def _prep_edges(ei):
    e = ei.shape[1] + N
    nch = -(-e // (NS * CH))
    nch = -(-nch // SB) * SB
    epad = NS * CH * nch
    loop = jnp.arange(N, dtype=jnp.int32)
    src = jnp.concatenate(
        [ei[0].astype(jnp.int32), loop,
         jnp.full((epad - e,), NP - 1, jnp.int32)])
    dst = jnp.concatenate(
        [ei[1].astype(jnp.int32), loop,
         jnp.full((epad - e,), NP - 1, jnp.int32)])
    return src.reshape(NS, nch, CH), dst.reshape(NS, nch, CH), nch


def kernel(x_raw, edge_industry, edge_universe, W_enc, b_enc, gamma, beta,
           W1, att_src1, att_dst1, bias1, W2, att_src2, att_dst2, bias2,
           Wf1, bf1, Wf2, bf2, Wa, ba):
    y, stats, attn_weights, fe8 = _tc1(x_raw, W_enc, b_enc, Wa, ba)
    c, h1, aa1, m1 = _tc2(y, stats, gamma, beta, W1, att_src1, att_dst1)

    src1, dst1, nch1 = _prep_edges(edge_industry)
    src2, dst2, nch2 = _prep_edges(edge_universe)

    def _sc_inputs(h, aa, m):
        hp = jnp.pad(h, ((0, NP - N), (0, 0)))
        hs = jnp.stack([hp[:, :HID], hp[:, HID:]])
        aap = jnp.pad(aa, ((0, NP - N), (0, 0)))
        aa2c = jnp.stack(
            [jnp.stack([aap[:, hd], aap[:, 2 + hd]], axis=-1).reshape(-1)
             for hd in range(HEADS)])
        m2c = jnp.broadcast_to(m[0, :16][None], (2, 16))
        return hs, aa2c, m2c

    h1p, aat1, m1c = _sc_inputs(h1, aa1, m1)
    nd1 = _sc_gat(h1p, aat1, m1c, src1, dst1, nch1)

    cb, h2, aa2, m2 = _tc3(nd1[:, :N], bias1, c, W2, att_src2, att_dst2)

    h2p, aat2, m2c = _sc_inputs(h2, aa2, m2)
    nd2 = _sc_gat(h2p, aat2, m2c, src2, dst2, nch2)

    df8 = _tc4(nd2[:, :N], bias2, c, cb, Wf1, bf1,
               Wf2.reshape(1, HID), bf2)[0]

    deep_factor = df8[:, 0]
    factor_estimate = fe8[:, 0]
    return (deep_factor, factor_estimate, attn_weights)
